# Initial kernel scaffold; baseline (speedup 1.0000x reference)
import jax
import jax.numpy as jnp
from jax.experimental import pallas as pl


def kernel(h, x, edges, vel, edge_attr, params):
    raise NotImplementedError("write your pallas kernel here")


# speedup vs baseline: 2.4619x; 2.4619x over previous
import functools

import jax
import jax.numpy as jnp
from jax import lax
from jax.experimental import pallas as pl
from jax.experimental.pallas import tpu as pltpu
from jax.experimental.pallas import tpu_sc as plsc

N = 10000
E = 320000
H = 128
XW = 16
PROJ = 64

NC = 2
NS = 16
NW = NC * NS
CHUNK = 128
CHUNKS_PER_W = E // (NW * CHUNK)
BASE_COVER = NW * CHUNKS_PER_W * CHUNK
EXTRA_CHUNKS = (E - BASE_COVER) // CHUNK

NROWS_PER_TILE = N // NS
ZROWS = 125

_MESH = plsc.VectorSubcoreMesh(
    core_axis_name="c", subcore_axis_name="s", num_cores=NC, num_subcores=NS
)
_SC_PARAMS = pltpu.CompilerParams(use_tc_tiling_on_sc=False)


def _lrelu(v):
    return jnp.where(v >= 0.0, v, 0.2 * v)


@functools.partial(
    pl.kernel,
    mesh=_MESH,
    out_type=[
        jax.ShapeDtypeStruct((E, H), jnp.float32),
        jax.ShapeDtypeStruct((E, H), jnp.float32),
        jax.ShapeDtypeStruct((E, XW), jnp.float32),
        jax.ShapeDtypeStruct((E, XW), jnp.float32),
    ],
    scratch_types=[
        pltpu.VMEM((2, CHUNK), jnp.int32),
        pltpu.VMEM((CHUNK, H), jnp.float32),
        pltpu.VMEM((CHUNK, H), jnp.float32),
        pltpu.VMEM((CHUNK, XW), jnp.float32),
        pltpu.VMEM((CHUNK, XW), jnp.float32),
        pltpu.SemaphoreType.DMA,
    ],
    compiler_params=_SC_PARAMS,
)
def _sc_gather(rtab, ctab, xtab, edges, gr, gc, gxr, gxc,
               idx_v, bufr, bufc, bufxr, bufxc, sem):
    cid = lax.axis_index("c")
    sid = lax.axis_index("s")
    wid = sid * NC + cid

    def do_chunk(base):
        pltpu.sync_copy(edges.at[0, pl.ds(base, CHUNK)], idx_v.at[0])
        pltpu.sync_copy(edges.at[1, pl.ds(base, CHUNK)], idx_v.at[1])
        g0 = pltpu.async_copy(rtab.at[idx_v.at[0]], bufr, sem)
        g1 = pltpu.async_copy(ctab.at[idx_v.at[1]], bufc, sem)
        g2 = pltpu.async_copy(xtab.at[idx_v.at[0]], bufxr, sem)
        g3 = pltpu.async_copy(xtab.at[idx_v.at[1]], bufxc, sem)
        g0.wait()
        g1.wait()
        g2.wait()
        g3.wait()
        pltpu.sync_copy(bufr, gr.at[pl.ds(base, CHUNK)])
        pltpu.sync_copy(bufc, gc.at[pl.ds(base, CHUNK)])
        pltpu.sync_copy(bufxr, gxr.at[pl.ds(base, CHUNK)])
        pltpu.sync_copy(bufxc, gxc.at[pl.ds(base, CHUNK)])

    base0 = wid * (CHUNKS_PER_W * CHUNK)

    def body(i, _):
        do_chunk(base0 + i * CHUNK)
        return 0

    lax.fori_loop(0, CHUNKS_PER_W, body, 0)

    @pl.when(wid < EXTRA_CHUNKS)
    def _():
        do_chunk(BASE_COVER + wid * CHUNK)


@functools.partial(
    pl.kernel,
    mesh=_MESH,
    out_type=[
        jax.ShapeDtypeStruct((NC, N, H), jnp.float32),
        jax.ShapeDtypeStruct((NC, N, XW), jnp.float32),
    ],
    scratch_types=[
        pltpu.VMEM((2, CHUNK), jnp.int32),
        pltpu.VMEM((CHUNK, H), jnp.float32),
        pltpu.VMEM((CHUNK, XW), jnp.float32),
        pltpu.VMEM((ZROWS, H), jnp.float32),
        pltpu.VMEM((ZROWS, XW), jnp.float32),
        pltpu.VMEM_SHARED((N, H), jnp.float32),
        pltpu.VMEM_SHARED((N, XW), jnp.float32),
        pltpu.SemaphoreType.DMA,
    ],
    compiler_params=_SC_PARAMS,
)
def _sc_scatter(ee, tt, edges, outm, outt,
                idx_v, bufe, buft, zbe, zbt, accm, acct, sem):
    cid = lax.axis_index("c")
    sid = lax.axis_index("s")
    wid = sid * NC + cid

    def zrow(i, _):
        r = i // (H // 16)
        o = (i % (H // 16)) * 16
        zbe[r, pl.ds(o, 16)] = jnp.zeros((16,), jnp.float32)
        return 0

    lax.fori_loop(0, ZROWS * (H // 16), zrow, 0)

    def zrowt(i, _):
        zbt[i, pl.ds(0, 16)] = jnp.zeros((16,), jnp.float32)
        return 0

    lax.fori_loop(0, ZROWS, zrowt, 0)

    def zcopy(k, _):
        r0 = sid * NROWS_PER_TILE + k * ZROWS
        pltpu.sync_copy(zbe, accm.at[pl.ds(r0, ZROWS)])
        pltpu.sync_copy(zbt, acct.at[pl.ds(r0, ZROWS)])
        return 0

    lax.fori_loop(0, NROWS_PER_TILE // ZROWS, zcopy, 0)
    plsc.subcore_barrier()

    def do_chunk(base):
        pltpu.sync_copy(edges.at[0, pl.ds(base, CHUNK)], idx_v.at[0])
        pltpu.sync_copy(ee.at[pl.ds(base, CHUNK)], bufe)
        pltpu.sync_copy(tt.at[pl.ds(base, CHUNK)], buft)
        pltpu.sync_copy(bufe, accm.at[idx_v.at[0]], add=True)
        pltpu.sync_copy(buft, acct.at[idx_v.at[0]], add=True)

    base0 = wid * (CHUNKS_PER_W * CHUNK)

    def body(i, _):
        do_chunk(base0 + i * CHUNK)
        return 0

    lax.fori_loop(0, CHUNKS_PER_W, body, 0)

    @pl.when(wid < EXTRA_CHUNKS)
    def _():
        do_chunk(BASE_COVER + wid * CHUNK)

    plsc.subcore_barrier()

    def drain(k, _):
        r0 = sid * NROWS_PER_TILE + k * ZROWS
        pltpu.sync_copy(accm.at[pl.ds(r0, ZROWS)], outm.at[cid, pl.ds(r0, ZROWS)])
        pltpu.sync_copy(acct.at[pl.ds(r0, ZROWS)], outt.at[cid, pl.ds(r0, ZROWS)])
        return 0

    lax.fori_loop(0, NROWS_PER_TILE // ZROWS, drain, 0)


_BE = 2560


def _edge_body(gr, gc, gxr, gxc, ea, w1e, w1r, w2, b2, wc1, bc1, wc2r,
               ee, tt):
    d = gxr[...] - gxc[...]
    rad = jnp.sum(d * d, axis=1, keepdims=True)
    s = gr[...] + gc[...] + rad * w1r[...] + jnp.dot(
        ea[...], w1e[...], preferred_element_type=jnp.float32,
        precision=lax.Precision.HIGHEST)
    e1 = _lrelu(s)
    e2 = _lrelu(jnp.dot(e1, w2[...], preferred_element_type=jnp.float32,
                        precision=lax.Precision.HIGHEST) + b2[...])
    c1 = _lrelu(jnp.dot(e2, wc1[...], preferred_element_type=jnp.float32,
                        precision=lax.Precision.HIGHEST) + bc1[...])
    c = jnp.sum(c1 * wc2r[...], axis=1, keepdims=True)
    t = jnp.clip(d * c, -100.0, 100.0)
    lane = lax.broadcasted_iota(jnp.int32, t.shape, 1)
    t = jnp.where(lane == 3, 1.0, t)
    ee[...] = e2
    tt[...] = t


def _tc_edge(gr, gc, gxr, gxc, ea, w1e, w1r, w2, b2, wc1, bc1, wc2r):
    grid = E // _BE
    hspec = pl.BlockSpec((_BE, H), lambda i: (i, 0))
    xspec = pl.BlockSpec((_BE, XW), lambda i: (i, 0))
    full = lambda a: pl.BlockSpec(a.shape, lambda i: (0,) * a.ndim)
    return pl.pallas_call(
        _edge_body,
        grid=grid,
        in_specs=[
            hspec, hspec, xspec, xspec,
            pl.BlockSpec((_BE, 16), lambda i: (i, 0)),
            full(w1e), full(w1r), full(w2), full(b2), full(wc1), full(bc1),
            full(wc2r),
        ],
        out_specs=[hspec, xspec],
        out_shape=[
            jax.ShapeDtypeStruct((E, H), jnp.float32),
            jax.ShapeDtypeStruct((E, XW), jnp.float32),
        ],
        compiler_params=pltpu.CompilerParams(
            dimension_semantics=("arbitrary",)),
    )(gr, gc, gxr, gxc, ea, w1e, w1r, w2, b2, wc1, bc1, wc2r)


_BN = 2000


def _dot(a, b):
    return jnp.dot(a, b, preferred_element_type=jnp.float32,
                   precision=lax.Precision.HIGHEST)


def _init_body(h, xp, we, be, w1a, w1b, b1, hout, rout, cout, xout):
    h0 = _dot(h[...], we[...]) + be[...]
    hout[...] = h0
    rout[...] = _dot(h0, w1a[...]) + b1[...]
    cout[...] = _dot(h0, w1b[...])
    xout[...] = xp[...]


def _tc_init(h, xp, we, be, w1a, w1b, b1):
    grid = N // _BN
    full = lambda a: pl.BlockSpec(a.shape, lambda i: (0,) * a.ndim)
    hspec = pl.BlockSpec((_BN, H), lambda i: (i, 0))
    xspec = pl.BlockSpec((_BN, XW), lambda i: (i, 0))
    return pl.pallas_call(
        _init_body,
        grid=grid,
        in_specs=[hspec, xspec, full(we), full(be), full(w1a), full(w1b),
                  full(b1)],
        out_specs=[hspec, hspec, hspec, xspec],
        out_shape=[
            jax.ShapeDtypeStruct((N, H), jnp.float32),
            jax.ShapeDtypeStruct((N, H), jnp.float32),
            jax.ShapeDtypeStruct((N, H), jnp.float32),
            jax.ShapeDtypeStruct((N, XW), jnp.float32),
        ],
        compiler_params=pltpu.CompilerParams(
            dimension_semantics=("arbitrary",)),
    )(h, xp, we, be, w1a, w1b, b1)


def _node_common(h, xold, pm, pt):
    m = pm[0] + pm[1]
    tacc = pt[0] + pt[1]
    cnt = tacc[:, 3:4]
    scale = 1.0 / jnp.maximum(cnt, 1.0)
    lane = lax.broadcasted_iota(jnp.int32, tacc.shape, 1)
    agg = jnp.where(lane < 3, tacc * scale, 0.0)
    return m, xold + agg


def _node_body(h, xold, pm, pt, wn1a, wn1b, bn1, wn2, bn2, w1a, w1b, b1,
               hout, rout, cout, xout):
    m, xnew = _node_common(h[...], xold[...], pm[...], pt[...])
    nh1 = _lrelu(_dot(h[...], wn1a[...]) + _dot(m, wn1b[...]) + bn1[...])
    hnew = h[...] + _dot(nh1, wn2[...]) + bn2[...]
    hout[...] = hnew
    rout[...] = _dot(hnew, w1a[...]) + b1[...]
    cout[...] = _dot(hnew, w1b[...])
    xout[...] = xnew


def _tc_node(h, xold, pm, pt, wn1a, wn1b, bn1, wn2, bn2, w1a, w1b, b1):
    grid = N // _BN
    full = lambda a: pl.BlockSpec(a.shape, lambda i: (0,) * a.ndim)
    hspec = pl.BlockSpec((_BN, H), lambda i: (i, 0))
    xspec = pl.BlockSpec((_BN, XW), lambda i: (i, 0))
    return pl.pallas_call(
        _node_body,
        grid=grid,
        in_specs=[
            hspec, xspec,
            pl.BlockSpec((NC, _BN, H), lambda i: (0, i, 0)),
            pl.BlockSpec((NC, _BN, XW), lambda i: (0, i, 0)),
            full(wn1a), full(wn1b), full(bn1), full(wn2), full(bn2),
            full(w1a), full(w1b), full(b1),
        ],
        out_specs=[hspec, hspec, hspec, xspec],
        out_shape=[
            jax.ShapeDtypeStruct((N, H), jnp.float32),
            jax.ShapeDtypeStruct((N, H), jnp.float32),
            jax.ShapeDtypeStruct((N, H), jnp.float32),
            jax.ShapeDtypeStruct((N, XW), jnp.float32),
        ],
        compiler_params=pltpu.CompilerParams(
            dimension_semantics=("arbitrary",)),
    )(h, xold, pm, pt, wn1a, wn1b, bn1, wn2, bn2, w1a, w1b, b1)


def _node_last_body(h, xold, pm, pt, wn1a, wn1b, bn1, wn2, bn2, wp, bp,
                    hout, xout):
    m, xnew = _node_common(h[...], xold[...], pm[...], pt[...])
    nh1 = _lrelu(_dot(h[...], wn1a[...]) + _dot(m, wn1b[...]) + bn1[...])
    hnew = h[...] + _dot(nh1, wn2[...]) + bn2[...]
    hout[...] = _dot(hnew, wp[...]) + bp[...]
    xout[...] = xnew


def _tc_node_last(h, xold, pm, pt, wn1a, wn1b, bn1, wn2, bn2, wp, bp):
    grid = N // _BN
    full = lambda a: pl.BlockSpec(a.shape, lambda i: (0,) * a.ndim)
    hspec = pl.BlockSpec((_BN, H), lambda i: (i, 0))
    xspec = pl.BlockSpec((_BN, XW), lambda i: (i, 0))
    return pl.pallas_call(
        _node_last_body,
        grid=grid,
        in_specs=[
            hspec, xspec,
            pl.BlockSpec((NC, _BN, H), lambda i: (0, i, 0)),
            pl.BlockSpec((NC, _BN, XW), lambda i: (0, i, 0)),
            full(wn1a), full(wn1b), full(bn1), full(wn2), full(bn2),
            full(wp), full(bp),
        ],
        out_specs=[
            pl.BlockSpec((_BN, PROJ), lambda i: (i, 0)),
            xspec,
        ],
        out_shape=[
            jax.ShapeDtypeStruct((N, PROJ), jnp.float32),
            jax.ShapeDtypeStruct((N, XW), jnp.float32),
        ],
        compiler_params=pltpu.CompilerParams(
            dimension_semantics=("arbitrary",)),
    )(h, xold, pm, pt, wn1a, wn1b, bn1, wn2, bn2, wp, bp)


def kernel(h, x, edges, vel, edge_attr, params):
    edges = edges.astype(jnp.int32)
    xp = jnp.pad(x.astype(jnp.float32), ((0, 0), (0, XW - 3)))

    def row(v):
        return v.reshape(1, -1)

    lw = []
    for lp in params["layers"]:
        w1 = lp["edge_mlp"][0]["w"]
        lw.append(dict(
            w1a=w1[:H], w1b=w1[H:2 * H], w1r=w1[2 * H:2 * H + 1],
            w1e=w1[2 * H + 1:], b1=row(lp["edge_mlp"][0]["b"]),
            w2=lp["edge_mlp"][1]["w"], b2=row(lp["edge_mlp"][1]["b"]),
            wc1=lp["coord_mlp"][0]["w"], bc1=row(lp["coord_mlp"][0]["b"]),
            wc2r=lp["coord_mlp"][1]["w"].reshape(1, H),
            wn1a=lp["node_mlp"][0]["w"][:H], wn1b=lp["node_mlp"][0]["w"][H:],
            bn1=row(lp["node_mlp"][0]["b"]),
            wn2=lp["node_mlp"][1]["w"], bn2=row(lp["node_mlp"][1]["b"]),
        ))

    hcur, rtab, ctab, xcur = _tc_init(
        h, xp, params["embed"]["w"], row(params["embed"]["b"]),
        lw[0]["w1a"], lw[0]["w1b"], lw[0]["b1"])

    for l in range(len(lw)):
        w = lw[l]
        gr, gc, gxr, gxc = _sc_gather(rtab, ctab, xcur, edges)
        ee, tt = _tc_edge(gr, gc, gxr, gxc, edge_attr, w["w1e"], w["w1r"],
                          w["w2"], w["b2"], w["wc1"], w["bc1"], w["wc2r"])
        pm, pt = _sc_scatter(ee, tt, edges)
        if l + 1 < len(lw):
            nw = lw[l + 1]
            hcur, rtab, ctab, xcur = _tc_node(
                hcur, xcur, pm, pt, w["wn1a"], w["wn1b"], w["bn1"], w["wn2"],
                w["bn2"], nw["w1a"], nw["w1b"], nw["b1"])
        else:
            hp, xout = _tc_node_last(
                hcur, xcur, pm, pt, w["wn1a"], w["wn1b"], w["bn1"], w["wn2"],
                w["bn2"], params["proj"]["w"], row(params["proj"]["b"]))

    return (hp, xout[:, :3], vel)


# speedup vs baseline: 2.5576x; 1.0389x over previous
import functools

import jax
import jax.numpy as jnp
from jax import lax
from jax.experimental import pallas as pl
from jax.experimental.pallas import tpu as pltpu
from jax.experimental.pallas import tpu_sc as plsc

N = 10000
E = 320000
H = 128
XW = 16
PROJ = 64

NC = 2
NS = 16
NW = NC * NS
CHUNK = 128
CHUNKS_PER_W = E // (NW * CHUNK)
BASE_COVER = NW * CHUNKS_PER_W * CHUNK
EXTRA_CHUNKS = (E - BASE_COVER) // CHUNK

NHALF = N // NC
NHP = NHALF + 8
ZROWS = 50
NDC = NHALF // ZROWS
NCHUNKS = E // CHUNK
CPT = NCHUNKS // NS
EXTRA_SC = NCHUNKS - CPT * NS

_MESH = plsc.VectorSubcoreMesh(
    core_axis_name="c", subcore_axis_name="s", num_cores=NC, num_subcores=NS
)
_SC_PARAMS = pltpu.CompilerParams(use_tc_tiling_on_sc=False)


def _lrelu(v):
    return jnp.where(v >= 0.0, v, 0.2 * v)


NQ = CHUNKS_PER_W // 4 + 1


def _gather_stream(tabh, tabx, idx2d, outh, outx, base0, bh, bx, sems):
    def g_descs(u, ci):
        return (
            pltpu.make_async_copy(tabh.at[idx2d.at[ci]], bh.at[u], sems[u]),
            pltpu.make_async_copy(tabx.at[idx2d.at[ci]], bx.at[u], sems[u]),
        )

    def w_descs(u, base):
        return (
            pltpu.make_async_copy(bh.at[u], outh.at[pl.ds(base, CHUNK)], sems[u]),
            pltpu.make_async_copy(bx.at[u], outx.at[pl.ds(base, CHUNK)], sems[u]),
        )

    def step(u, i):
        @pl.when(jnp.logical_and(i >= 4, i < CHUNKS_PER_W))
        def _():
            for d in w_descs(u, base0 + (i - 4) * CHUNK):
                d.wait()

        @pl.when(i < CHUNKS_PER_W)
        def _():
            for d in g_descs(u, i):
                d.start()

        @pl.when(jnp.logical_and(i >= 1, i <= CHUNKS_PER_W))
        def _():
            for d in g_descs((u - 1) % 4, i - 1):
                d.wait()
            for d in w_descs((u - 1) % 4, base0 + (i - 1) * CHUNK):
                d.start()

    def body(q, _):
        for u in range(4):
            step(u, q * 4 + u)
        return 0

    lax.fori_loop(0, NQ, body, 0)
    for k in range(4):
        ci = CHUNKS_PER_W - 4 + k
        for d in w_descs(ci % 4, base0 + ci * CHUNK):
            d.wait()


@functools.partial(
    pl.kernel,
    mesh=_MESH,
    out_type=[
        jax.ShapeDtypeStruct((E, H), jnp.float32),
        jax.ShapeDtypeStruct((E, H), jnp.float32),
        jax.ShapeDtypeStruct((E, XW), jnp.float32),
        jax.ShapeDtypeStruct((E, XW), jnp.float32),
    ],
    scratch_types=[
        pltpu.VMEM((CHUNKS_PER_W + 1, CHUNK), jnp.int32),
        pltpu.VMEM((CHUNKS_PER_W + 1, CHUNK), jnp.int32),
        pltpu.VMEM((4, CHUNK, H), jnp.float32),
        pltpu.VMEM((4, CHUNK, XW), jnp.float32),
        pltpu.SemaphoreType.DMA,
        pltpu.SemaphoreType.DMA,
        pltpu.SemaphoreType.DMA,
        pltpu.SemaphoreType.DMA,
    ],
    compiler_params=_SC_PARAMS,
)
def _sc_gather(rtab, ctab, xtab, edges3, gr, gc, gxr, gxc,
               idxr, idxc, bh, bx, s0, s1, s2, s3):
    cid = lax.axis_index("c")
    sid = lax.axis_index("s")
    wid = sid * NC + cid
    sems = (s0, s1, s2, s3)

    cb0 = wid * CHUNKS_PER_W
    base0 = cb0 * CHUNK
    pltpu.sync_copy(edges3.at[0, pl.ds(cb0, CHUNKS_PER_W)],
                    idxr.at[pl.ds(0, CHUNKS_PER_W)])
    pltpu.sync_copy(edges3.at[1, pl.ds(cb0, CHUNKS_PER_W)],
                    idxc.at[pl.ds(0, CHUNKS_PER_W)])

    _gather_stream(rtab, xtab, idxr, gr, gxr, base0, bh, bx, sems)
    _gather_stream(ctab, xtab, idxc, gc, gxc, base0, bh, bx, sems)

    @pl.when(wid < EXTRA_CHUNKS)
    def _():
        cbx = BASE_COVER // CHUNK + wid
        basex = cbx * CHUNK
        pltpu.sync_copy(edges3.at[0, pl.ds(cbx, 1)],
                        idxr.at[pl.ds(CHUNKS_PER_W, 1)])
        pltpu.sync_copy(edges3.at[1, pl.ds(cbx, 1)],
                        idxc.at[pl.ds(CHUNKS_PER_W, 1)])
        for tab, idx, outh, outx in (
            (rtab, idxr, gr, gxr), (ctab, idxc, gc, gxc)):
            g0 = pltpu.make_async_copy(tab.at[idx.at[CHUNKS_PER_W]],
                                       bh.at[0], s0)
            g1 = pltpu.make_async_copy(xtab.at[idx.at[CHUNKS_PER_W]],
                                       bx.at[0], s0)
            g0.start(); g1.start(); g0.wait(); g1.wait()
            pltpu.sync_copy(bh.at[0], outh.at[pl.ds(basex, CHUNK)])
            pltpu.sync_copy(bx.at[0], outx.at[pl.ds(basex, CHUNK)])


@functools.partial(
    pl.kernel,
    mesh=_MESH,
    out_type=[
        jax.ShapeDtypeStruct((N, H), jnp.float32),
        jax.ShapeDtypeStruct((N, XW), jnp.float32),
    ],
    scratch_types=[
        pltpu.VMEM((CPT + 1, CHUNK), jnp.int32),
        pltpu.VMEM((1, CHUNK), jnp.int32),
        pltpu.VMEM((2, CHUNK, H), jnp.float32),
        pltpu.VMEM((2, CHUNK, XW), jnp.float32),
        pltpu.VMEM((ZROWS, H), jnp.float32),
        pltpu.VMEM((ZROWS, XW), jnp.float32),
        pltpu.VMEM_SHARED((NHP, H), jnp.float32),
        pltpu.VMEM_SHARED((NHP, XW), jnp.float32),
        pltpu.SemaphoreType.DMA,
        pltpu.SemaphoreType.DMA,
    ],
    compiler_params=_SC_PARAMS,
)
def _sc_scatter(ee, tt, edges3, outm, outt,
                idxr, idxc_cur, be, bt, zbe, zbt, accm, acct, s0, s1):
    cid = lax.axis_index("c")
    sid = lax.axis_index("s")
    sems = (s0, s1)

    def zrow(i, _):
        r = i // (H // 16)
        o = (i % (H // 16)) * 16
        zbe[r, pl.ds(o, 16)] = jnp.zeros((16,), jnp.float32)
        return 0

    lax.fori_loop(0, ZROWS * (H // 16), zrow, 0)

    def zrowt(i, _):
        zbt[i, pl.ds(0, 16)] = jnp.zeros((16,), jnp.float32)
        return 0

    lax.fori_loop(0, ZROWS, zrowt, 0)

    def zcopy(k, _):
        c = sid + k * NS

        @pl.when(c < NDC)
        def _():
            pltpu.sync_copy(zbe, accm.at[pl.ds(c * ZROWS, ZROWS)])
            pltpu.sync_copy(zbt, acct.at[pl.ds(c * ZROWS, ZROWS)])
        return 0

    lax.fori_loop(0, (NDC + NS - 1) // NS, zcopy, 0)

    @pl.when(sid == NS - 1)
    def _():
        pltpu.sync_copy(zbe.at[pl.ds(0, 8)], accm.at[pl.ds(NHALF, 8)])
        pltpu.sync_copy(zbt.at[pl.ds(0, 8)], acct.at[pl.ds(NHALF, 8)])

    cb0 = sid * CPT
    base0 = cb0 * CHUNK
    pltpu.sync_copy(edges3.at[0, pl.ds(cb0, CPT)], idxr.at[pl.ds(0, CPT)])

    lo = cid * NHALF

    def localize(i, _):
        r = i // (CHUNK // 16)
        o = (i % (CHUNK // 16)) * 16
        v = idxr[r, pl.ds(o, 16)] - lo
        ok = jnp.logical_and(v >= 0, v < NHALF)
        idxr[r, pl.ds(o, 16)] = jnp.where(ok, v, NHALF)
        return 0

    lax.fori_loop(0, CPT * (CHUNK // 16), localize, 0)
    plsc.subcore_barrier()

    def l_descs(u, base):
        return (
            pltpu.make_async_copy(ee.at[pl.ds(base, CHUNK)], be.at[u], sems[u]),
            pltpu.make_async_copy(tt.at[pl.ds(base, CHUNK)], bt.at[u], sems[u]),
        )

    def s_adds(u, ci):
        for g in range(CHUNK // 16):
            idxc_cur[0, pl.ds(g * 16, 16)] = idxr[ci, pl.ds(g * 16, 16)]
        pltpu.sync_copy(be.at[u], accm.at[idxc_cur.at[0]], add=True)
        pltpu.sync_copy(bt.at[u], acct.at[idxc_cur.at[0]], add=True)

    def step(u, i):
        @pl.when(i < CPT)
        def _():
            for d in l_descs(u, base0 + i * CHUNK):
                d.start()

        @pl.when(jnp.logical_and(i >= 1, i <= CPT))
        def _():
            for d in l_descs((u - 1) % 2, base0 + (i - 1) * CHUNK):
                d.wait()
            s_adds((u - 1) % 2, i - 1)

    def body(q, _):
        for u in range(2):
            step(u, q * 2 + u)
        return 0

    lax.fori_loop(0, CPT // 2 + 1, body, 0)

    @pl.when(sid < EXTRA_SC)
    def _():
        cbx = CPT * NS + sid
        basex = cbx * CHUNK
        pltpu.sync_copy(edges3.at[0, pl.ds(cbx, 1)], idxr.at[pl.ds(CPT, 1)])

        def localx(g, _):
            v = idxr[CPT, pl.ds(g * 16, 16)] - lo
            ok = jnp.logical_and(v >= 0, v < NHALF)
            idxr[CPT, pl.ds(g * 16, 16)] = jnp.where(ok, v, NHALF)
            return 0

        lax.fori_loop(0, CHUNK // 16, localx, 0)
        for d in l_descs(0, basex):
            d.start()
        for d in l_descs(0, basex):
            d.wait()
        s_adds(0, CPT)

    plsc.subcore_barrier()

    def drain(k, _):
        c = sid + k * NS

        @pl.when(c < NDC)
        def _():
            r0 = c * ZROWS
            pltpu.sync_copy(accm.at[pl.ds(r0, ZROWS)],
                            outm.at[pl.ds(lo + r0, ZROWS)])
            pltpu.sync_copy(acct.at[pl.ds(r0, ZROWS)],
                            outt.at[pl.ds(lo + r0, ZROWS)])
        return 0

    lax.fori_loop(0, (NDC + NS - 1) // NS, drain, 0)


_BE = 2560


def _edge_body(gr, gc, gxr, gxc, ea, w1e, w1r, w2, b2, wc1, bc1, wc2r,
               ee, tt):
    d = gxr[...] - gxc[...]
    rad = jnp.sum(d * d, axis=1, keepdims=True)
    s = gr[...] + gc[...] + rad * w1r[...] + jnp.dot(
        ea[...], w1e[...], preferred_element_type=jnp.float32,
        precision=lax.Precision.HIGHEST)
    e1 = _lrelu(s)
    e2 = _lrelu(jnp.dot(e1, w2[...], preferred_element_type=jnp.float32,
                        precision=lax.Precision.HIGHEST) + b2[...])
    c1 = _lrelu(jnp.dot(e2, wc1[...], preferred_element_type=jnp.float32,
                        precision=lax.Precision.HIGHEST) + bc1[...])
    c = jnp.sum(c1 * wc2r[...], axis=1, keepdims=True)
    t = jnp.clip(d * c, -100.0, 100.0)
    lane = lax.broadcasted_iota(jnp.int32, t.shape, 1)
    t = jnp.where(lane == 3, 1.0, t)
    ee[...] = e2
    tt[...] = t


def _tc_edge(gr, gc, gxr, gxc, ea, w1e, w1r, w2, b2, wc1, bc1, wc2r):
    grid = E // _BE
    hspec = pl.BlockSpec((_BE, H), lambda i: (i, 0))
    xspec = pl.BlockSpec((_BE, XW), lambda i: (i, 0))
    full = lambda a: pl.BlockSpec(a.shape, lambda i: (0,) * a.ndim)
    return pl.pallas_call(
        _edge_body,
        grid=grid,
        in_specs=[
            hspec, hspec, xspec, xspec,
            pl.BlockSpec((_BE, 16), lambda i: (i, 0)),
            full(w1e), full(w1r), full(w2), full(b2), full(wc1), full(bc1),
            full(wc2r),
        ],
        out_specs=[hspec, xspec],
        out_shape=[
            jax.ShapeDtypeStruct((E, H), jnp.float32),
            jax.ShapeDtypeStruct((E, XW), jnp.float32),
        ],
        compiler_params=pltpu.CompilerParams(
            dimension_semantics=("arbitrary",)),
    )(gr, gc, gxr, gxc, ea, w1e, w1r, w2, b2, wc1, bc1, wc2r)


_BN = 2000


def _dot(a, b):
    return jnp.dot(a, b, preferred_element_type=jnp.float32,
                   precision=lax.Precision.HIGHEST)


def _init_body(h, xp, we, be, w1a, w1b, b1, hout, rout, cout, xout):
    h0 = _dot(h[...], we[...]) + be[...]
    hout[...] = h0
    rout[...] = _dot(h0, w1a[...]) + b1[...]
    cout[...] = _dot(h0, w1b[...])
    xout[...] = xp[...]


def _tc_init(h, xp, we, be, w1a, w1b, b1):
    grid = N // _BN
    full = lambda a: pl.BlockSpec(a.shape, lambda i: (0,) * a.ndim)
    hspec = pl.BlockSpec((_BN, H), lambda i: (i, 0))
    xspec = pl.BlockSpec((_BN, XW), lambda i: (i, 0))
    return pl.pallas_call(
        _init_body,
        grid=grid,
        in_specs=[hspec, xspec, full(we), full(be), full(w1a), full(w1b),
                  full(b1)],
        out_specs=[hspec, hspec, hspec, xspec],
        out_shape=[
            jax.ShapeDtypeStruct((N, H), jnp.float32),
            jax.ShapeDtypeStruct((N, H), jnp.float32),
            jax.ShapeDtypeStruct((N, H), jnp.float32),
            jax.ShapeDtypeStruct((N, XW), jnp.float32),
        ],
        compiler_params=pltpu.CompilerParams(
            dimension_semantics=("arbitrary",)),
    )(h, xp, we, be, w1a, w1b, b1)


def _node_common(h, xold, pm, pt):
    m = pm
    tacc = pt
    cnt = tacc[:, 3:4]
    scale = 1.0 / jnp.maximum(cnt, 1.0)
    lane = lax.broadcasted_iota(jnp.int32, tacc.shape, 1)
    agg = jnp.where(lane < 3, tacc * scale, 0.0)
    return m, xold + agg


def _node_body(h, xold, pm, pt, wn1a, wn1b, bn1, wn2, bn2, w1a, w1b, b1,
               hout, rout, cout, xout):
    m, xnew = _node_common(h[...], xold[...], pm[...], pt[...])
    nh1 = _lrelu(_dot(h[...], wn1a[...]) + _dot(m, wn1b[...]) + bn1[...])
    hnew = h[...] + _dot(nh1, wn2[...]) + bn2[...]
    hout[...] = hnew
    rout[...] = _dot(hnew, w1a[...]) + b1[...]
    cout[...] = _dot(hnew, w1b[...])
    xout[...] = xnew


def _tc_node(h, xold, pm, pt, wn1a, wn1b, bn1, wn2, bn2, w1a, w1b, b1):
    grid = N // _BN
    full = lambda a: pl.BlockSpec(a.shape, lambda i: (0,) * a.ndim)
    hspec = pl.BlockSpec((_BN, H), lambda i: (i, 0))
    xspec = pl.BlockSpec((_BN, XW), lambda i: (i, 0))
    return pl.pallas_call(
        _node_body,
        grid=grid,
        in_specs=[
            hspec, xspec, hspec, xspec,
            full(wn1a), full(wn1b), full(bn1), full(wn2), full(bn2),
            full(w1a), full(w1b), full(b1),
        ],
        out_specs=[hspec, hspec, hspec, xspec],
        out_shape=[
            jax.ShapeDtypeStruct((N, H), jnp.float32),
            jax.ShapeDtypeStruct((N, H), jnp.float32),
            jax.ShapeDtypeStruct((N, H), jnp.float32),
            jax.ShapeDtypeStruct((N, XW), jnp.float32),
        ],
        compiler_params=pltpu.CompilerParams(
            dimension_semantics=("arbitrary",)),
    )(h, xold, pm, pt, wn1a, wn1b, bn1, wn2, bn2, w1a, w1b, b1)


def _node_last_body(h, xold, pm, pt, wn1a, wn1b, bn1, wn2, bn2, wp, bp,
                    hout, xout):
    m, xnew = _node_common(h[...], xold[...], pm[...], pt[...])
    nh1 = _lrelu(_dot(h[...], wn1a[...]) + _dot(m, wn1b[...]) + bn1[...])
    hnew = h[...] + _dot(nh1, wn2[...]) + bn2[...]
    hout[...] = _dot(hnew, wp[...]) + bp[...]
    xout[...] = xnew


def _tc_node_last(h, xold, pm, pt, wn1a, wn1b, bn1, wn2, bn2, wp, bp):
    grid = N // _BN
    full = lambda a: pl.BlockSpec(a.shape, lambda i: (0,) * a.ndim)
    hspec = pl.BlockSpec((_BN, H), lambda i: (i, 0))
    xspec = pl.BlockSpec((_BN, XW), lambda i: (i, 0))
    return pl.pallas_call(
        _node_last_body,
        grid=grid,
        in_specs=[
            hspec, xspec, hspec, xspec,
            full(wn1a), full(wn1b), full(bn1), full(wn2), full(bn2),
            full(wp), full(bp),
        ],
        out_specs=[
            pl.BlockSpec((_BN, PROJ), lambda i: (i, 0)),
            xspec,
        ],
        out_shape=[
            jax.ShapeDtypeStruct((N, PROJ), jnp.float32),
            jax.ShapeDtypeStruct((N, XW), jnp.float32),
        ],
        compiler_params=pltpu.CompilerParams(
            dimension_semantics=("arbitrary",)),
    )(h, xold, pm, pt, wn1a, wn1b, bn1, wn2, bn2, wp, bp)


def kernel(h, x, edges, vel, edge_attr, params):
    edges3 = edges.astype(jnp.int32).reshape(2, E // CHUNK, CHUNK)
    xp = jnp.pad(x.astype(jnp.float32), ((0, 0), (0, XW - 3)))

    def row(v):
        return v.reshape(1, -1)

    lw = []
    for lp in params["layers"]:
        w1 = lp["edge_mlp"][0]["w"]
        lw.append(dict(
            w1a=w1[:H], w1b=w1[H:2 * H], w1r=w1[2 * H:2 * H + 1],
            w1e=w1[2 * H + 1:], b1=row(lp["edge_mlp"][0]["b"]),
            w2=lp["edge_mlp"][1]["w"], b2=row(lp["edge_mlp"][1]["b"]),
            wc1=lp["coord_mlp"][0]["w"], bc1=row(lp["coord_mlp"][0]["b"]),
            wc2r=lp["coord_mlp"][1]["w"].reshape(1, H),
            wn1a=lp["node_mlp"][0]["w"][:H], wn1b=lp["node_mlp"][0]["w"][H:],
            bn1=row(lp["node_mlp"][0]["b"]),
            wn2=lp["node_mlp"][1]["w"], bn2=row(lp["node_mlp"][1]["b"]),
        ))

    hcur, rtab, ctab, xcur = _tc_init(
        h, xp, params["embed"]["w"], row(params["embed"]["b"]),
        lw[0]["w1a"], lw[0]["w1b"], lw[0]["b1"])

    for l in range(len(lw)):
        w = lw[l]
        gr, gc, gxr, gxc = _sc_gather(rtab, ctab, xcur, edges3)
        ee, tt = _tc_edge(gr, gc, gxr, gxc, edge_attr, w["w1e"], w["w1r"],
                          w["w2"], w["b2"], w["wc1"], w["bc1"], w["wc2r"])
        pm, pt = _sc_scatter(ee, tt, edges3)
        if l + 1 < len(lw):
            nw = lw[l + 1]
            hcur, rtab, ctab, xcur = _tc_node(
                hcur, xcur, pm, pt, w["wn1a"], w["wn1b"], w["bn1"], w["wn2"],
                w["bn2"], nw["w1a"], nw["w1b"], nw["b1"])
        else:
            hp, xout = _tc_node_last(
                hcur, xcur, pm, pt, w["wn1a"], w["wn1b"], w["bn1"], w["wn2"],
                w["bn2"], params["proj"]["w"], row(params["proj"]["b"]))

    return (hp, xout[:, :3], vel)


# speedup vs baseline: 2.8438x; 1.1119x over previous
import functools

import jax
import jax.numpy as jnp
from jax import lax
from jax.experimental import pallas as pl
from jax.experimental.pallas import tpu as pltpu
from jax.experimental.pallas import tpu_sc as plsc

N = 10000
E = 320000
H = 128
XW = 16
PROJ = 64

NC = 2
NS = 16
NW = NC * NS
CHUNK = 128
CHUNKS_PER_W = E // (NW * CHUNK)
BASE_COVER = NW * CHUNKS_PER_W * CHUNK
EXTRA_CHUNKS = (E - BASE_COVER) // CHUNK

ZROWS = 25
NRT = N // NS
CH_S = 64
CPS = E // (NW * CH_S)
COVER_S = NW * CPS * CH_S
EXTRA_S = (E - COVER_S) // CH_S

_MESH = plsc.VectorSubcoreMesh(
    core_axis_name="c", subcore_axis_name="s", num_cores=NC, num_subcores=NS
)
_SC_PARAMS = pltpu.CompilerParams(use_tc_tiling_on_sc=False)


def _lrelu(v):
    return jnp.where(v >= 0.0, v, 0.2 * v)


NQ = CHUNKS_PER_W // 4 + 1


def _gather_stream(tabh, tabx, idx2d, outh, outx, base0, bh, bx, sems):
    def g_descs(u, ci):
        return (
            pltpu.make_async_copy(tabh.at[idx2d.at[ci]], bh.at[u], sems[u]),
            pltpu.make_async_copy(tabx.at[idx2d.at[ci]], bx.at[u], sems[u]),
        )

    def w_descs(u, base):
        return (
            pltpu.make_async_copy(bh.at[u], outh.at[pl.ds(base, CHUNK)], sems[u]),
            pltpu.make_async_copy(bx.at[u], outx.at[pl.ds(base, CHUNK)], sems[u]),
        )

    def step(u, i):
        @pl.when(jnp.logical_and(i >= 4, i < CHUNKS_PER_W))
        def _():
            for d in w_descs(u, base0 + (i - 4) * CHUNK):
                d.wait()

        @pl.when(i < CHUNKS_PER_W)
        def _():
            for d in g_descs(u, i):
                d.start()

        @pl.when(jnp.logical_and(i >= 1, i <= CHUNKS_PER_W))
        def _():
            for d in g_descs((u - 1) % 4, i - 1):
                d.wait()
            for d in w_descs((u - 1) % 4, base0 + (i - 1) * CHUNK):
                d.start()

    def body(q, _):
        for u in range(4):
            step(u, q * 4 + u)
        return 0

    lax.fori_loop(0, NQ, body, 0)
    for k in range(4):
        ci = CHUNKS_PER_W - 4 + k
        for d in w_descs(ci % 4, base0 + ci * CHUNK):
            d.wait()


@functools.partial(
    pl.kernel,
    mesh=_MESH,
    out_type=[
        jax.ShapeDtypeStruct((E, H), jnp.float32),
        jax.ShapeDtypeStruct((E, H), jnp.float32),
        jax.ShapeDtypeStruct((E, XW), jnp.float32),
        jax.ShapeDtypeStruct((E, XW), jnp.float32),
    ],
    scratch_types=[
        pltpu.VMEM((CHUNKS_PER_W + 1, CHUNK), jnp.int32),
        pltpu.VMEM((CHUNKS_PER_W + 1, CHUNK), jnp.int32),
        pltpu.VMEM((4, CHUNK, H), jnp.float32),
        pltpu.VMEM((4, CHUNK, XW), jnp.float32),
        pltpu.SemaphoreType.DMA,
        pltpu.SemaphoreType.DMA,
        pltpu.SemaphoreType.DMA,
        pltpu.SemaphoreType.DMA,
    ],
    compiler_params=_SC_PARAMS,
)
def _sc_gather(rtab, ctab, xtab, edges3, gr, gc, gxr, gxc,
               idxr, idxc, bh, bx, s0, s1, s2, s3):
    cid = lax.axis_index("c")
    sid = lax.axis_index("s")
    wid = sid * NC + cid
    sems = (s0, s1, s2, s3)

    cb0 = wid * CHUNKS_PER_W
    base0 = cb0 * CHUNK
    pltpu.sync_copy(edges3.at[0, pl.ds(cb0, CHUNKS_PER_W)],
                    idxr.at[pl.ds(0, CHUNKS_PER_W)])
    pltpu.sync_copy(edges3.at[1, pl.ds(cb0, CHUNKS_PER_W)],
                    idxc.at[pl.ds(0, CHUNKS_PER_W)])

    _gather_stream(rtab, xtab, idxr, gr, gxr, base0, bh, bx, sems)
    _gather_stream(ctab, xtab, idxc, gc, gxc, base0, bh, bx, sems)

    @pl.when(wid < EXTRA_CHUNKS)
    def _():
        cbx = BASE_COVER // CHUNK + wid
        basex = cbx * CHUNK
        pltpu.sync_copy(edges3.at[0, pl.ds(cbx, 1)],
                        idxr.at[pl.ds(CHUNKS_PER_W, 1)])
        pltpu.sync_copy(edges3.at[1, pl.ds(cbx, 1)],
                        idxc.at[pl.ds(CHUNKS_PER_W, 1)])
        for tab, idx, outh, outx in (
            (rtab, idxr, gr, gxr), (ctab, idxc, gc, gxc)):
            g0 = pltpu.make_async_copy(tab.at[idx.at[CHUNKS_PER_W]],
                                       bh.at[0], s0)
            g1 = pltpu.make_async_copy(xtab.at[idx.at[CHUNKS_PER_W]],
                                       bx.at[0], s0)
            g0.start(); g1.start(); g0.wait(); g1.wait()
            pltpu.sync_copy(bh.at[0], outh.at[pl.ds(basex, CHUNK)])
            pltpu.sync_copy(bx.at[0], outx.at[pl.ds(basex, CHUNK)])


@functools.partial(
    pl.kernel,
    mesh=_MESH,
    out_type=[
        jax.ShapeDtypeStruct((NC, N, H), jnp.float32),
        jax.ShapeDtypeStruct((NC, N, XW), jnp.float32),
    ],
    scratch_types=[
        pltpu.VMEM((CPS + 1, CH_S), jnp.int32),
        pltpu.VMEM((2, CH_S, H), jnp.float32),
        pltpu.VMEM((2, CH_S, XW), jnp.float32),
        pltpu.VMEM((ZROWS, H), jnp.float32),
        pltpu.VMEM((ZROWS, XW), jnp.float32),
        pltpu.VMEM_SHARED((N, H), jnp.float32),
        pltpu.VMEM_SHARED((N, XW), jnp.float32),
        pltpu.SemaphoreType.DMA,
        pltpu.SemaphoreType.DMA,
    ],
    compiler_params=_SC_PARAMS,
)
def _sc_scatter(ee, tt, edges64, outm, outt,
                idxr, be, bt, zbe, zbt, accm, acct, s0, s1):
    cid = lax.axis_index("c")
    sid = lax.axis_index("s")
    wid = sid * NC + cid
    sems = (s0, s1)

    def zrow(i, _):
        r = i // (H // 16)
        o = (i % (H // 16)) * 16
        zbe[r, pl.ds(o, 16)] = jnp.zeros((16,), jnp.float32)
        return 0

    lax.fori_loop(0, ZROWS * (H // 16), zrow, 0)

    def zrowt(i, _):
        zbt[i, pl.ds(0, 16)] = jnp.zeros((16,), jnp.float32)
        return 0

    lax.fori_loop(0, ZROWS, zrowt, 0)

    def zcopy(k, _):
        r0 = sid * NRT + k * ZROWS
        pltpu.sync_copy(zbe, accm.at[pl.ds(r0, ZROWS)])
        pltpu.sync_copy(zbt, acct.at[pl.ds(r0, ZROWS)])
        return 0

    lax.fori_loop(0, NRT // ZROWS, zcopy, 0)

    cb0 = wid * CPS
    base0 = cb0 * CH_S
    pltpu.sync_copy(edges64.at[0, pl.ds(cb0, CPS)], idxr.at[pl.ds(0, CPS)])
    plsc.subcore_barrier()

    def l_descs(u, base):
        return (
            pltpu.make_async_copy(ee.at[pl.ds(base, CH_S)], be.at[u], sems[u]),
            pltpu.make_async_copy(tt.at[pl.ds(base, CH_S)], bt.at[u], sems[u]),
        )

    def s_descs(u, ci):
        return (
            pltpu.make_async_copy(be.at[u], accm.at[idxr.at[ci]], sems[u]),
            pltpu.make_async_copy(bt.at[u], acct.at[idxr.at[ci]], sems[u]),
        )

    def step(u, i):
        @pl.when(jnp.logical_and(i >= 2, i < CPS))
        def _():
            for d in s_descs(u, i - 2):
                d.wait()

        @pl.when(i < CPS)
        def _():
            for d in l_descs(u, base0 + i * CH_S):
                d.start()

        @pl.when(jnp.logical_and(i >= 1, i <= CPS))
        def _():
            for d in l_descs((u - 1) % 2, base0 + (i - 1) * CH_S):
                d.wait()
            for d in s_descs((u - 1) % 2, i - 1):
                d.start(add=True)

    def body(q, _):
        for u in range(2):
            step(u, q * 2 + u)
        return 0

    lax.fori_loop(0, CPS // 2 + 1, body, 0)
    for k in range(2):
        ci = CPS - 2 + k
        for d in s_descs(ci % 2, ci):
            d.wait()

    @pl.when(wid < EXTRA_S)
    def _():
        cbx = CPS * NW + wid
        basex = cbx * CH_S
        pltpu.sync_copy(edges64.at[0, pl.ds(cbx, 1)], idxr.at[pl.ds(CPS, 1)])
        for d in l_descs(0, basex):
            d.start()
        for d in l_descs(0, basex):
            d.wait()
        for d in s_descs(0, CPS):
            d.start(add=True)
        for d in s_descs(0, CPS):
            d.wait()

    plsc.subcore_barrier()

    def drain(k, _):
        r0 = sid * NRT + k * ZROWS
        pltpu.sync_copy(accm.at[pl.ds(r0, ZROWS)],
                        outm.at[cid, pl.ds(r0, ZROWS)])
        pltpu.sync_copy(acct.at[pl.ds(r0, ZROWS)],
                        outt.at[cid, pl.ds(r0, ZROWS)])
        return 0

    lax.fori_loop(0, NRT // ZROWS, drain, 0)


_BE = 2560


def _edge_body(gr, gc, gxr, gxc, ea, w1e, w1r, w2, b2, wc1, bc1, wc2r,
               ee, tt):
    d = gxr[...] - gxc[...]
    rad = jnp.sum(d * d, axis=1, keepdims=True)
    s = gr[...] + gc[...] + rad * w1r[...] + jnp.dot(
        ea[...], w1e[...], preferred_element_type=jnp.float32,
        precision=lax.Precision.HIGHEST)
    e1 = _lrelu(s)
    e2 = _lrelu(jnp.dot(e1, w2[...], preferred_element_type=jnp.float32,
                        precision=lax.Precision.HIGHEST) + b2[...])
    c1 = _lrelu(jnp.dot(e2, wc1[...], preferred_element_type=jnp.float32,
                        precision=lax.Precision.HIGHEST) + bc1[...])
    c = jnp.sum(c1 * wc2r[...], axis=1, keepdims=True)
    t = jnp.clip(d * c, -100.0, 100.0)
    lane = lax.broadcasted_iota(jnp.int32, t.shape, 1)
    t = jnp.where(lane == 3, 1.0, t)
    ee[...] = e2
    tt[...] = t


def _tc_edge(gr, gc, gxr, gxc, ea, w1e, w1r, w2, b2, wc1, bc1, wc2r):
    grid = E // _BE
    hspec = pl.BlockSpec((_BE, H), lambda i: (i, 0))
    xspec = pl.BlockSpec((_BE, XW), lambda i: (i, 0))
    full = lambda a: pl.BlockSpec(a.shape, lambda i: (0,) * a.ndim)
    return pl.pallas_call(
        _edge_body,
        grid=grid,
        in_specs=[
            hspec, hspec, xspec, xspec,
            pl.BlockSpec((_BE, 16), lambda i: (i, 0)),
            full(w1e), full(w1r), full(w2), full(b2), full(wc1), full(bc1),
            full(wc2r),
        ],
        out_specs=[hspec, xspec],
        out_shape=[
            jax.ShapeDtypeStruct((E, H), jnp.float32),
            jax.ShapeDtypeStruct((E, XW), jnp.float32),
        ],
        compiler_params=pltpu.CompilerParams(
            dimension_semantics=("arbitrary",)),
    )(gr, gc, gxr, gxc, ea, w1e, w1r, w2, b2, wc1, bc1, wc2r)


_BN = 2000


def _dot(a, b):
    return jnp.dot(a, b, preferred_element_type=jnp.float32,
                   precision=lax.Precision.HIGHEST)


def _init_body(h, xp, we, be, w1a, w1b, b1, hout, rout, cout, xout):
    h0 = _dot(h[...], we[...]) + be[...]
    hout[...] = h0
    rout[...] = _dot(h0, w1a[...]) + b1[...]
    cout[...] = _dot(h0, w1b[...])
    xout[...] = xp[...]


def _tc_init(h, xp, we, be, w1a, w1b, b1):
    grid = N // _BN
    full = lambda a: pl.BlockSpec(a.shape, lambda i: (0,) * a.ndim)
    hspec = pl.BlockSpec((_BN, H), lambda i: (i, 0))
    xspec = pl.BlockSpec((_BN, XW), lambda i: (i, 0))
    return pl.pallas_call(
        _init_body,
        grid=grid,
        in_specs=[hspec, xspec, full(we), full(be), full(w1a), full(w1b),
                  full(b1)],
        out_specs=[hspec, hspec, hspec, xspec],
        out_shape=[
            jax.ShapeDtypeStruct((N, H), jnp.float32),
            jax.ShapeDtypeStruct((N, H), jnp.float32),
            jax.ShapeDtypeStruct((N, H), jnp.float32),
            jax.ShapeDtypeStruct((N, XW), jnp.float32),
        ],
        compiler_params=pltpu.CompilerParams(
            dimension_semantics=("arbitrary",)),
    )(h, xp, we, be, w1a, w1b, b1)


def _node_common(h, xold, pm, pt):
    m = pm[0] + pm[1]
    tacc = pt[0] + pt[1]
    cnt = tacc[:, 3:4]
    scale = 1.0 / jnp.maximum(cnt, 1.0)
    lane = lax.broadcasted_iota(jnp.int32, tacc.shape, 1)
    agg = jnp.where(lane < 3, tacc * scale, 0.0)
    return m, xold + agg


def _node_body(h, xold, pm, pt, wn1a, wn1b, bn1, wn2, bn2, w1a, w1b, b1,
               hout, rout, cout, xout):
    m, xnew = _node_common(h[...], xold[...], pm[...], pt[...])
    nh1 = _lrelu(_dot(h[...], wn1a[...]) + _dot(m, wn1b[...]) + bn1[...])
    hnew = h[...] + _dot(nh1, wn2[...]) + bn2[...]
    hout[...] = hnew
    rout[...] = _dot(hnew, w1a[...]) + b1[...]
    cout[...] = _dot(hnew, w1b[...])
    xout[...] = xnew


def _tc_node(h, xold, pm, pt, wn1a, wn1b, bn1, wn2, bn2, w1a, w1b, b1):
    grid = N // _BN
    full = lambda a: pl.BlockSpec(a.shape, lambda i: (0,) * a.ndim)
    hspec = pl.BlockSpec((_BN, H), lambda i: (i, 0))
    xspec = pl.BlockSpec((_BN, XW), lambda i: (i, 0))
    return pl.pallas_call(
        _node_body,
        grid=grid,
        in_specs=[
            hspec, xspec,
            pl.BlockSpec((NC, _BN, H), lambda i: (0, i, 0)),
            pl.BlockSpec((NC, _BN, XW), lambda i: (0, i, 0)),
            full(wn1a), full(wn1b), full(bn1), full(wn2), full(bn2),
            full(w1a), full(w1b), full(b1),
        ],
        out_specs=[hspec, hspec, hspec, xspec],
        out_shape=[
            jax.ShapeDtypeStruct((N, H), jnp.float32),
            jax.ShapeDtypeStruct((N, H), jnp.float32),
            jax.ShapeDtypeStruct((N, H), jnp.float32),
            jax.ShapeDtypeStruct((N, XW), jnp.float32),
        ],
        compiler_params=pltpu.CompilerParams(
            dimension_semantics=("arbitrary",)),
    )(h, xold, pm, pt, wn1a, wn1b, bn1, wn2, bn2, w1a, w1b, b1)


def _node_last_body(h, xold, pm, pt, wn1a, wn1b, bn1, wn2, bn2, wp, bp,
                    hout, xout):
    m, xnew = _node_common(h[...], xold[...], pm[...], pt[...])
    nh1 = _lrelu(_dot(h[...], wn1a[...]) + _dot(m, wn1b[...]) + bn1[...])
    hnew = h[...] + _dot(nh1, wn2[...]) + bn2[...]
    hout[...] = _dot(hnew, wp[...]) + bp[...]
    xout[...] = xnew


def _tc_node_last(h, xold, pm, pt, wn1a, wn1b, bn1, wn2, bn2, wp, bp):
    grid = N // _BN
    full = lambda a: pl.BlockSpec(a.shape, lambda i: (0,) * a.ndim)
    hspec = pl.BlockSpec((_BN, H), lambda i: (i, 0))
    xspec = pl.BlockSpec((_BN, XW), lambda i: (i, 0))
    return pl.pallas_call(
        _node_last_body,
        grid=grid,
        in_specs=[
            hspec, xspec,
            pl.BlockSpec((NC, _BN, H), lambda i: (0, i, 0)),
            pl.BlockSpec((NC, _BN, XW), lambda i: (0, i, 0)),
            full(wn1a), full(wn1b), full(bn1), full(wn2), full(bn2),
            full(wp), full(bp),
        ],
        out_specs=[
            pl.BlockSpec((_BN, PROJ), lambda i: (i, 0)),
            xspec,
        ],
        out_shape=[
            jax.ShapeDtypeStruct((N, PROJ), jnp.float32),
            jax.ShapeDtypeStruct((N, XW), jnp.float32),
        ],
        compiler_params=pltpu.CompilerParams(
            dimension_semantics=("arbitrary",)),
    )(h, xold, pm, pt, wn1a, wn1b, bn1, wn2, bn2, wp, bp)


def kernel(h, x, edges, vel, edge_attr, params):
    edges = edges.astype(jnp.int32)
    edges3 = edges.reshape(2, E // CHUNK, CHUNK)
    edges64 = edges.reshape(2, E // CH_S, CH_S)
    xp = jnp.pad(x.astype(jnp.float32), ((0, 0), (0, XW - 3)))

    def row(v):
        return v.reshape(1, -1)

    lw = []
    for lp in params["layers"]:
        w1 = lp["edge_mlp"][0]["w"]
        lw.append(dict(
            w1a=w1[:H], w1b=w1[H:2 * H], w1r=w1[2 * H:2 * H + 1],
            w1e=w1[2 * H + 1:], b1=row(lp["edge_mlp"][0]["b"]),
            w2=lp["edge_mlp"][1]["w"], b2=row(lp["edge_mlp"][1]["b"]),
            wc1=lp["coord_mlp"][0]["w"], bc1=row(lp["coord_mlp"][0]["b"]),
            wc2r=lp["coord_mlp"][1]["w"].reshape(1, H),
            wn1a=lp["node_mlp"][0]["w"][:H], wn1b=lp["node_mlp"][0]["w"][H:],
            bn1=row(lp["node_mlp"][0]["b"]),
            wn2=lp["node_mlp"][1]["w"], bn2=row(lp["node_mlp"][1]["b"]),
        ))

    hcur, rtab, ctab, xcur = _tc_init(
        h, xp, params["embed"]["w"], row(params["embed"]["b"]),
        lw[0]["w1a"], lw[0]["w1b"], lw[0]["b1"])

    for l in range(len(lw)):
        w = lw[l]
        gr, gc, gxr, gxc = _sc_gather(rtab, ctab, xcur, edges3)
        ee, tt = _tc_edge(gr, gc, gxr, gxc, edge_attr, w["w1e"], w["w1r"],
                          w["w2"], w["b2"], w["wc1"], w["bc1"], w["wc2r"])
        pm, pt = _sc_scatter(ee, tt, edges64)
        if l + 1 < len(lw):
            nw = lw[l + 1]
            hcur, rtab, ctab, xcur = _tc_node(
                hcur, xcur, pm, pt, w["wn1a"], w["wn1b"], w["bn1"], w["wn2"],
                w["bn2"], nw["w1a"], nw["w1b"], nw["b1"])
        else:
            hp, xout = _tc_node_last(
                hcur, xcur, pm, pt, w["wn1a"], w["wn1b"], w["bn1"], w["wn2"],
                w["bn2"], params["proj"]["w"], row(params["proj"]["b"]))

    return (hp, xout[:, :3], vel)


# speedup vs baseline: 3.6882x; 1.2969x over previous
import functools

import jax
import jax.numpy as jnp
from jax import lax
from jax.experimental import pallas as pl
from jax.experimental.pallas import tpu as pltpu
from jax.experimental.pallas import tpu_sc as plsc

N = 10000
E = 320000
H = 128
XW = 16
PROJ = 64

NC = 2
NS = 16
NW = NC * NS
CHUNK = 128
CHUNKS_PER_W = E // (NW * CHUNK)
BASE_COVER = NW * CHUNKS_PER_W * CHUNK
EXTRA_CHUNKS = (E - BASE_COVER) // CHUNK

ZROWS = 25
NRT = N // NS
CH_S = 64
CPS = E // (NW * CH_S)
COVER_S = NW * CPS * CH_S
EXTRA_S = (E - COVER_S) // CH_S

_MESH = plsc.VectorSubcoreMesh(
    core_axis_name="c", subcore_axis_name="s", num_cores=NC, num_subcores=NS
)
_SC_PARAMS = pltpu.CompilerParams(use_tc_tiling_on_sc=False)


def _lrelu(v):
    return jnp.where(v >= 0.0, v, 0.2 * v)


NQ = CHUNKS_PER_W // 4 + 1


def _gather_stream(tabh, tabx, idx2d, outh, outx, base0, bh, bx, sems):
    def g_descs(u, ci):
        return (
            pltpu.make_async_copy(tabh.at[idx2d.at[ci]], bh.at[u], sems[u]),
            pltpu.make_async_copy(tabx.at[idx2d.at[ci]], bx.at[u], sems[u]),
        )

    def w_descs(u, base):
        return (
            pltpu.make_async_copy(bh.at[u], outh.at[pl.ds(base, CHUNK)], sems[u]),
            pltpu.make_async_copy(bx.at[u], outx.at[pl.ds(base, CHUNK)], sems[u]),
        )

    def step(u, i):
        @pl.when(jnp.logical_and(i >= 4, i < CHUNKS_PER_W))
        def _():
            for d in w_descs(u, base0 + (i - 4) * CHUNK):
                d.wait()

        @pl.when(i < CHUNKS_PER_W)
        def _():
            for d in g_descs(u, i):
                d.start()

        @pl.when(jnp.logical_and(i >= 1, i <= CHUNKS_PER_W))
        def _():
            for d in g_descs((u - 1) % 4, i - 1):
                d.wait()
            for d in w_descs((u - 1) % 4, base0 + (i - 1) * CHUNK):
                d.start()

    def body(q, _):
        for u in range(4):
            step(u, q * 4 + u)
        return 0

    lax.fori_loop(0, NQ, body, 0)
    for k in range(4):
        ci = CHUNKS_PER_W - 4 + k
        for d in w_descs(ci % 4, base0 + ci * CHUNK):
            d.wait()


@functools.partial(
    pl.kernel,
    mesh=_MESH,
    out_type=[
        jax.ShapeDtypeStruct((E, H), jnp.float32),
        jax.ShapeDtypeStruct((E, H), jnp.float32),
        jax.ShapeDtypeStruct((E, XW), jnp.float32),
        jax.ShapeDtypeStruct((E, XW), jnp.float32),
    ],
    scratch_types=[
        pltpu.VMEM((CHUNKS_PER_W + 1, CHUNK), jnp.int32),
        pltpu.VMEM((CHUNKS_PER_W + 1, CHUNK), jnp.int32),
        pltpu.VMEM((4, CHUNK, H), jnp.float32),
        pltpu.VMEM((4, CHUNK, XW), jnp.float32),
        pltpu.SemaphoreType.DMA,
        pltpu.SemaphoreType.DMA,
        pltpu.SemaphoreType.DMA,
        pltpu.SemaphoreType.DMA,
    ],
    compiler_params=_SC_PARAMS,
)
def _sc_gather(rtab, ctab, xtab, edges3, gr, gc, gxr, gxc,
               idxr, idxc, bh, bx, s0, s1, s2, s3):
    cid = lax.axis_index("c")
    sid = lax.axis_index("s")
    wid = sid * NC + cid
    sems = (s0, s1, s2, s3)

    cb0 = wid * CHUNKS_PER_W
    base0 = cb0 * CHUNK
    pltpu.sync_copy(edges3.at[0, pl.ds(cb0, CHUNKS_PER_W)],
                    idxr.at[pl.ds(0, CHUNKS_PER_W)])
    pltpu.sync_copy(edges3.at[1, pl.ds(cb0, CHUNKS_PER_W)],
                    idxc.at[pl.ds(0, CHUNKS_PER_W)])

    _gather_stream(rtab, xtab, idxr, gr, gxr, base0, bh, bx, sems)
    _gather_stream(ctab, xtab, idxc, gc, gxc, base0, bh, bx, sems)

    @pl.when(wid < EXTRA_CHUNKS)
    def _():
        cbx = BASE_COVER // CHUNK + wid
        basex = cbx * CHUNK
        pltpu.sync_copy(edges3.at[0, pl.ds(cbx, 1)],
                        idxr.at[pl.ds(CHUNKS_PER_W, 1)])
        pltpu.sync_copy(edges3.at[1, pl.ds(cbx, 1)],
                        idxc.at[pl.ds(CHUNKS_PER_W, 1)])
        for tab, idx, outh, outx in (
            (rtab, idxr, gr, gxr), (ctab, idxc, gc, gxc)):
            g0 = pltpu.make_async_copy(tab.at[idx.at[CHUNKS_PER_W]],
                                       bh.at[0], s0)
            g1 = pltpu.make_async_copy(xtab.at[idx.at[CHUNKS_PER_W]],
                                       bx.at[0], s0)
            g0.start(); g1.start(); g0.wait(); g1.wait()
            pltpu.sync_copy(bh.at[0], outh.at[pl.ds(basex, CHUNK)])
            pltpu.sync_copy(bx.at[0], outx.at[pl.ds(basex, CHUNK)])


@functools.partial(
    pl.kernel,
    mesh=_MESH,
    out_type=[
        jax.ShapeDtypeStruct((NC, N, H), jnp.float32),
        jax.ShapeDtypeStruct((NC, N, XW), jnp.float32),
    ],
    scratch_types=[
        pltpu.VMEM((CPS + 1, CH_S), jnp.int32),
        pltpu.VMEM((2, CH_S, H), jnp.float32),
        pltpu.VMEM((2, CH_S, XW), jnp.float32),
        pltpu.VMEM((ZROWS, H), jnp.float32),
        pltpu.VMEM((ZROWS, XW), jnp.float32),
        pltpu.VMEM_SHARED((N, H), jnp.float32),
        pltpu.VMEM_SHARED((N, XW), jnp.float32),
        pltpu.SemaphoreType.DMA,
        pltpu.SemaphoreType.DMA,
    ],
    compiler_params=_SC_PARAMS,
)
def _sc_scatter(ee, tt, edges64, outm, outt,
                idxr, be, bt, zbe, zbt, accm, acct, s0, s1):
    cid = lax.axis_index("c")
    sid = lax.axis_index("s")
    wid = sid * NC + cid
    sems = (s0, s1)

    def zrow(i, _):
        r = i // (H // 16)
        o = (i % (H // 16)) * 16
        zbe[r, pl.ds(o, 16)] = jnp.zeros((16,), jnp.float32)
        return 0

    lax.fori_loop(0, ZROWS * (H // 16), zrow, 0)

    def zrowt(i, _):
        zbt[i, pl.ds(0, 16)] = jnp.zeros((16,), jnp.float32)
        return 0

    lax.fori_loop(0, ZROWS, zrowt, 0)

    def zcopy(k, _):
        r0 = sid * NRT + k * ZROWS
        pltpu.sync_copy(zbe, accm.at[pl.ds(r0, ZROWS)])
        pltpu.sync_copy(zbt, acct.at[pl.ds(r0, ZROWS)])
        return 0

    lax.fori_loop(0, NRT // ZROWS, zcopy, 0)

    cb0 = wid * CPS
    base0 = cb0 * CH_S
    pltpu.sync_copy(edges64.at[0, pl.ds(cb0, CPS)], idxr.at[pl.ds(0, CPS)])
    plsc.subcore_barrier()

    def l_descs(u, base):
        return (
            pltpu.make_async_copy(ee.at[pl.ds(base, CH_S)], be.at[u], sems[u]),
            pltpu.make_async_copy(tt.at[pl.ds(base, CH_S)], bt.at[u], sems[u]),
        )

    def s_descs(u, ci):
        return (
            pltpu.make_async_copy(be.at[u], accm.at[idxr.at[ci]], sems[u]),
            pltpu.make_async_copy(bt.at[u], acct.at[idxr.at[ci]], sems[u]),
        )

    def step(u, i):
        @pl.when(jnp.logical_and(i >= 2, i < CPS))
        def _():
            for d in s_descs(u, i - 2):
                d.wait()

        @pl.when(i < CPS)
        def _():
            for d in l_descs(u, base0 + i * CH_S):
                d.start()

        @pl.when(jnp.logical_and(i >= 1, i <= CPS))
        def _():
            for d in l_descs((u - 1) % 2, base0 + (i - 1) * CH_S):
                d.wait()
            for d in s_descs((u - 1) % 2, i - 1):
                d.start(add=True)

    def body(q, _):
        for u in range(2):
            step(u, q * 2 + u)
        return 0

    lax.fori_loop(0, CPS // 2 + 1, body, 0)
    for k in range(2):
        ci = CPS - 2 + k
        for d in s_descs(ci % 2, ci):
            d.wait()

    @pl.when(wid < EXTRA_S)
    def _():
        cbx = CPS * NW + wid
        basex = cbx * CH_S
        pltpu.sync_copy(edges64.at[0, pl.ds(cbx, 1)], idxr.at[pl.ds(CPS, 1)])
        for d in l_descs(0, basex):
            d.start()
        for d in l_descs(0, basex):
            d.wait()
        for d in s_descs(0, CPS):
            d.start(add=True)
        for d in s_descs(0, CPS):
            d.wait()

    plsc.subcore_barrier()

    def drain(k, _):
        r0 = sid * NRT + k * ZROWS
        pltpu.sync_copy(accm.at[pl.ds(r0, ZROWS)],
                        outm.at[cid, pl.ds(r0, ZROWS)])
        pltpu.sync_copy(acct.at[pl.ds(r0, ZROWS)],
                        outt.at[cid, pl.ds(r0, ZROWS)])
        return 0

    lax.fori_loop(0, NRT // ZROWS, drain, 0)


_BE = 2560


def _edge_body(gr, gc, gxr, gxc, ea, w1e, w1r, w2, b2, wc1, bc1, wc2r,
               ee, tt):
    d = gxr[...] - gxc[...]
    rad = jnp.sum(d * d, axis=1, keepdims=True)
    s = gr[...] + gc[...] + rad * w1r[...] + jnp.dot(
        ea[...], w1e[...], preferred_element_type=jnp.float32,
        precision=lax.Precision.DEFAULT)
    e1 = _lrelu(s)
    e2 = _lrelu(jnp.dot(e1, w2[...], preferred_element_type=jnp.float32,
                        precision=lax.Precision.DEFAULT) + b2[...])
    c1 = _lrelu(jnp.dot(e2, wc1[...], preferred_element_type=jnp.float32,
                        precision=lax.Precision.DEFAULT) + bc1[...])
    c = jnp.sum(c1 * wc2r[...], axis=1, keepdims=True)
    t = jnp.clip(d * c, -100.0, 100.0)
    lane = lax.broadcasted_iota(jnp.int32, t.shape, 1)
    t = jnp.where(lane == 3, 1.0, t)
    ee[...] = e2
    tt[...] = t


def _tc_edge(gr, gc, gxr, gxc, ea, w1e, w1r, w2, b2, wc1, bc1, wc2r):
    grid = E // _BE
    hspec = pl.BlockSpec((_BE, H), lambda i: (i, 0))
    xspec = pl.BlockSpec((_BE, XW), lambda i: (i, 0))
    full = lambda a: pl.BlockSpec(a.shape, lambda i: (0,) * a.ndim)
    return pl.pallas_call(
        _edge_body,
        grid=grid,
        in_specs=[
            hspec, hspec, xspec, xspec,
            pl.BlockSpec((_BE, 16), lambda i: (i, 0)),
            full(w1e), full(w1r), full(w2), full(b2), full(wc1), full(bc1),
            full(wc2r),
        ],
        out_specs=[hspec, xspec],
        out_shape=[
            jax.ShapeDtypeStruct((E, H), jnp.float32),
            jax.ShapeDtypeStruct((E, XW), jnp.float32),
        ],
        compiler_params=pltpu.CompilerParams(
            dimension_semantics=("arbitrary",)),
    )(gr, gc, gxr, gxc, ea, w1e, w1r, w2, b2, wc1, bc1, wc2r)


_BN = 2000


def _dot(a, b):
    return jnp.dot(a, b, preferred_element_type=jnp.float32,
                   precision=lax.Precision.DEFAULT)


def _init_body(h, xp, we, be, w1a, w1b, b1, hout, rout, cout, xout):
    h0 = _dot(h[...], we[...]) + be[...]
    hout[...] = h0
    rout[...] = _dot(h0, w1a[...]) + b1[...]
    cout[...] = _dot(h0, w1b[...])
    xout[...] = xp[...]


def _tc_init(h, xp, we, be, w1a, w1b, b1):
    grid = N // _BN
    full = lambda a: pl.BlockSpec(a.shape, lambda i: (0,) * a.ndim)
    hspec = pl.BlockSpec((_BN, H), lambda i: (i, 0))
    xspec = pl.BlockSpec((_BN, XW), lambda i: (i, 0))
    return pl.pallas_call(
        _init_body,
        grid=grid,
        in_specs=[hspec, xspec, full(we), full(be), full(w1a), full(w1b),
                  full(b1)],
        out_specs=[hspec, hspec, hspec, xspec],
        out_shape=[
            jax.ShapeDtypeStruct((N, H), jnp.float32),
            jax.ShapeDtypeStruct((N, H), jnp.float32),
            jax.ShapeDtypeStruct((N, H), jnp.float32),
            jax.ShapeDtypeStruct((N, XW), jnp.float32),
        ],
        compiler_params=pltpu.CompilerParams(
            dimension_semantics=("arbitrary",)),
    )(h, xp, we, be, w1a, w1b, b1)


def _node_common(h, xold, pm, pt):
    m = pm[0] + pm[1]
    tacc = pt[0] + pt[1]
    cnt = tacc[:, 3:4]
    scale = 1.0 / jnp.maximum(cnt, 1.0)
    lane = lax.broadcasted_iota(jnp.int32, tacc.shape, 1)
    agg = jnp.where(lane < 3, tacc * scale, 0.0)
    return m, xold + agg


def _node_body(h, xold, pm, pt, wn1a, wn1b, bn1, wn2, bn2, w1a, w1b, b1,
               hout, rout, cout, xout):
    m, xnew = _node_common(h[...], xold[...], pm[...], pt[...])
    nh1 = _lrelu(_dot(h[...], wn1a[...]) + _dot(m, wn1b[...]) + bn1[...])
    hnew = h[...] + _dot(nh1, wn2[...]) + bn2[...]
    hout[...] = hnew
    rout[...] = _dot(hnew, w1a[...]) + b1[...]
    cout[...] = _dot(hnew, w1b[...])
    xout[...] = xnew


def _tc_node(h, xold, pm, pt, wn1a, wn1b, bn1, wn2, bn2, w1a, w1b, b1):
    grid = N // _BN
    full = lambda a: pl.BlockSpec(a.shape, lambda i: (0,) * a.ndim)
    hspec = pl.BlockSpec((_BN, H), lambda i: (i, 0))
    xspec = pl.BlockSpec((_BN, XW), lambda i: (i, 0))
    return pl.pallas_call(
        _node_body,
        grid=grid,
        in_specs=[
            hspec, xspec,
            pl.BlockSpec((NC, _BN, H), lambda i: (0, i, 0)),
            pl.BlockSpec((NC, _BN, XW), lambda i: (0, i, 0)),
            full(wn1a), full(wn1b), full(bn1), full(wn2), full(bn2),
            full(w1a), full(w1b), full(b1),
        ],
        out_specs=[hspec, hspec, hspec, xspec],
        out_shape=[
            jax.ShapeDtypeStruct((N, H), jnp.float32),
            jax.ShapeDtypeStruct((N, H), jnp.float32),
            jax.ShapeDtypeStruct((N, H), jnp.float32),
            jax.ShapeDtypeStruct((N, XW), jnp.float32),
        ],
        compiler_params=pltpu.CompilerParams(
            dimension_semantics=("arbitrary",)),
    )(h, xold, pm, pt, wn1a, wn1b, bn1, wn2, bn2, w1a, w1b, b1)


def _node_last_body(h, xold, pm, pt, wn1a, wn1b, bn1, wn2, bn2, wp, bp,
                    hout, xout):
    m, xnew = _node_common(h[...], xold[...], pm[...], pt[...])
    nh1 = _lrelu(_dot(h[...], wn1a[...]) + _dot(m, wn1b[...]) + bn1[...])
    hnew = h[...] + _dot(nh1, wn2[...]) + bn2[...]
    hout[...] = _dot(hnew, wp[...]) + bp[...]
    xout[...] = xnew


def _tc_node_last(h, xold, pm, pt, wn1a, wn1b, bn1, wn2, bn2, wp, bp):
    grid = N // _BN
    full = lambda a: pl.BlockSpec(a.shape, lambda i: (0,) * a.ndim)
    hspec = pl.BlockSpec((_BN, H), lambda i: (i, 0))
    xspec = pl.BlockSpec((_BN, XW), lambda i: (i, 0))
    return pl.pallas_call(
        _node_last_body,
        grid=grid,
        in_specs=[
            hspec, xspec,
            pl.BlockSpec((NC, _BN, H), lambda i: (0, i, 0)),
            pl.BlockSpec((NC, _BN, XW), lambda i: (0, i, 0)),
            full(wn1a), full(wn1b), full(bn1), full(wn2), full(bn2),
            full(wp), full(bp),
        ],
        out_specs=[
            pl.BlockSpec((_BN, PROJ), lambda i: (i, 0)),
            xspec,
        ],
        out_shape=[
            jax.ShapeDtypeStruct((N, PROJ), jnp.float32),
            jax.ShapeDtypeStruct((N, XW), jnp.float32),
        ],
        compiler_params=pltpu.CompilerParams(
            dimension_semantics=("arbitrary",)),
    )(h, xold, pm, pt, wn1a, wn1b, bn1, wn2, bn2, wp, bp)


def kernel(h, x, edges, vel, edge_attr, params):
    edges = edges.astype(jnp.int32)
    edges3 = edges.reshape(2, E // CHUNK, CHUNK)
    edges64 = edges.reshape(2, E // CH_S, CH_S)
    xp = jnp.pad(x.astype(jnp.float32), ((0, 0), (0, XW - 3)))

    def row(v):
        return v.reshape(1, -1)

    lw = []
    for lp in params["layers"]:
        w1 = lp["edge_mlp"][0]["w"]
        lw.append(dict(
            w1a=w1[:H], w1b=w1[H:2 * H], w1r=w1[2 * H:2 * H + 1],
            w1e=w1[2 * H + 1:], b1=row(lp["edge_mlp"][0]["b"]),
            w2=lp["edge_mlp"][1]["w"], b2=row(lp["edge_mlp"][1]["b"]),
            wc1=lp["coord_mlp"][0]["w"], bc1=row(lp["coord_mlp"][0]["b"]),
            wc2r=lp["coord_mlp"][1]["w"].reshape(1, H),
            wn1a=lp["node_mlp"][0]["w"][:H], wn1b=lp["node_mlp"][0]["w"][H:],
            bn1=row(lp["node_mlp"][0]["b"]),
            wn2=lp["node_mlp"][1]["w"], bn2=row(lp["node_mlp"][1]["b"]),
        ))

    hcur, rtab, ctab, xcur = _tc_init(
        h, xp, params["embed"]["w"], row(params["embed"]["b"]),
        lw[0]["w1a"], lw[0]["w1b"], lw[0]["b1"])

    for l in range(len(lw)):
        w = lw[l]
        gr, gc, gxr, gxc = _sc_gather(rtab, ctab, xcur, edges3)
        ee, tt = _tc_edge(gr, gc, gxr, gxc, edge_attr, w["w1e"], w["w1r"],
                          w["w2"], w["b2"], w["wc1"], w["bc1"], w["wc2r"])
        pm, pt = _sc_scatter(ee, tt, edges64)
        if l + 1 < len(lw):
            nw = lw[l + 1]
            hcur, rtab, ctab, xcur = _tc_node(
                hcur, xcur, pm, pt, w["wn1a"], w["wn1b"], w["bn1"], w["wn2"],
                w["bn2"], nw["w1a"], nw["w1b"], nw["b1"])
        else:
            hp, xout = _tc_node_last(
                hcur, xcur, pm, pt, w["wn1a"], w["wn1b"], w["bn1"], w["wn2"],
                w["bn2"], params["proj"]["w"], row(params["proj"]["b"]))

    return (hp, xout[:, :3], vel)


# speedup vs baseline: 3.9316x; 1.0660x over previous
import functools

import jax
import jax.numpy as jnp
from jax import lax
from jax.experimental import pallas as pl
from jax.experimental.pallas import tpu as pltpu
from jax.experimental.pallas import tpu_sc as plsc

N = 10000
E = 320000
H = 128
XW = 16
PROJ = 64

NC = 2
NS = 16
NW = NC * NS
EH = E // 2
CHUNK = 128
CHUNKS_PER_W = EH // (NW * CHUNK)
BASE_COVER = NW * CHUNKS_PER_W * CHUNK
EXTRA_CHUNKS = (EH - BASE_COVER) // CHUNK

ZROWS = 25
NRT = N // NS
CH_S = 64
CPS = EH // (NW * CH_S)
COVER_S = NW * CPS * CH_S
EXTRA_S = (EH - COVER_S) // CH_S

_MESH = plsc.VectorSubcoreMesh(
    core_axis_name="c", subcore_axis_name="s", num_cores=NC, num_subcores=NS
)
_SC_PARAMS = pltpu.CompilerParams(use_tc_tiling_on_sc=False)


def _lrelu(v):
    return jnp.where(v >= 0.0, v, 0.2 * v)


NQ = CHUNKS_PER_W // 4 + 1


def _gather_stream(tabh, tabx, idx2d, outh, outx, base0, bh, bx, sems):
    def g_descs(u, ci):
        return (
            pltpu.make_async_copy(tabh.at[idx2d.at[ci]], bh.at[u], sems[u]),
            pltpu.make_async_copy(tabx.at[idx2d.at[ci]], bx.at[u], sems[u]),
        )

    def w_descs(u, base):
        return (
            pltpu.make_async_copy(bh.at[u], outh.at[pl.ds(base, CHUNK)], sems[u]),
            pltpu.make_async_copy(bx.at[u], outx.at[pl.ds(base, CHUNK)], sems[u]),
        )

    def step(u, i):
        @pl.when(jnp.logical_and(i >= 4, i < CHUNKS_PER_W))
        def _():
            for d in w_descs(u, base0 + (i - 4) * CHUNK):
                d.wait()

        @pl.when(i < CHUNKS_PER_W)
        def _():
            for d in g_descs(u, i):
                d.start()

        @pl.when(jnp.logical_and(i >= 1, i <= CHUNKS_PER_W))
        def _():
            for d in g_descs((u - 1) % 4, i - 1):
                d.wait()
            for d in w_descs((u - 1) % 4, base0 + (i - 1) * CHUNK):
                d.start()

    def body(q, _):
        for u in range(4):
            step(u, q * 4 + u)
        return 0

    lax.fori_loop(0, NQ, body, 0)
    for k in range(4):
        ci = CHUNKS_PER_W - 4 + k
        for d in w_descs(ci % 4, base0 + ci * CHUNK):
            d.wait()


@functools.partial(
    pl.kernel,
    mesh=_MESH,
    out_type=[
        jax.ShapeDtypeStruct((EH, H), jnp.float32),
        jax.ShapeDtypeStruct((EH, H), jnp.float32),
        jax.ShapeDtypeStruct((EH, XW), jnp.float32),
        jax.ShapeDtypeStruct((EH, XW), jnp.float32),
    ],
    scratch_types=[
        pltpu.VMEM((CHUNKS_PER_W + 1, CHUNK), jnp.int32),
        pltpu.VMEM((CHUNKS_PER_W + 1, CHUNK), jnp.int32),
        pltpu.VMEM((4, CHUNK, H), jnp.float32),
        pltpu.VMEM((4, CHUNK, XW), jnp.float32),
        pltpu.SemaphoreType.DMA,
        pltpu.SemaphoreType.DMA,
        pltpu.SemaphoreType.DMA,
        pltpu.SemaphoreType.DMA,
    ],
    compiler_params=_SC_PARAMS,
)
def _sc_gather(rtab, ctab, xtab, edges3, gr, gc, gxr, gxc,
               idxr, idxc, bh, bx, s0, s1, s2, s3):
    cid = lax.axis_index("c")
    sid = lax.axis_index("s")
    wid = sid * NC + cid
    sems = (s0, s1, s2, s3)

    cb0 = wid * CHUNKS_PER_W
    base0 = cb0 * CHUNK
    pltpu.sync_copy(edges3.at[0, pl.ds(cb0, CHUNKS_PER_W)],
                    idxr.at[pl.ds(0, CHUNKS_PER_W)])
    pltpu.sync_copy(edges3.at[1, pl.ds(cb0, CHUNKS_PER_W)],
                    idxc.at[pl.ds(0, CHUNKS_PER_W)])

    _gather_stream(rtab, xtab, idxr, gr, gxr, base0, bh, bx, sems)
    _gather_stream(ctab, xtab, idxc, gc, gxc, base0, bh, bx, sems)

    @pl.when(wid < EXTRA_CHUNKS)
    def _():
        cbx = BASE_COVER // CHUNK + wid
        basex = cbx * CHUNK
        pltpu.sync_copy(edges3.at[0, pl.ds(cbx, 1)],
                        idxr.at[pl.ds(CHUNKS_PER_W, 1)])
        pltpu.sync_copy(edges3.at[1, pl.ds(cbx, 1)],
                        idxc.at[pl.ds(CHUNKS_PER_W, 1)])
        for tab, idx, outh, outx in (
            (rtab, idxr, gr, gxr), (ctab, idxc, gc, gxc)):
            g0 = pltpu.make_async_copy(tab.at[idx.at[CHUNKS_PER_W]],
                                       bh.at[0], s0)
            g1 = pltpu.make_async_copy(xtab.at[idx.at[CHUNKS_PER_W]],
                                       bx.at[0], s0)
            g0.start(); g1.start(); g0.wait(); g1.wait()
            pltpu.sync_copy(bh.at[0], outh.at[pl.ds(basex, CHUNK)])
            pltpu.sync_copy(bx.at[0], outx.at[pl.ds(basex, CHUNK)])


@functools.partial(
    pl.kernel,
    mesh=_MESH,
    out_type=[
        jax.ShapeDtypeStruct((NC, N, H), jnp.float32),
        jax.ShapeDtypeStruct((NC, N, XW), jnp.float32),
    ],
    scratch_types=[
        pltpu.VMEM((CPS + 1, CH_S), jnp.int32),
        pltpu.VMEM((2, CH_S, H), jnp.float32),
        pltpu.VMEM((2, CH_S, XW), jnp.float32),
        pltpu.VMEM((ZROWS, H), jnp.float32),
        pltpu.VMEM((ZROWS, XW), jnp.float32),
        pltpu.VMEM_SHARED((N, H), jnp.float32),
        pltpu.VMEM_SHARED((N, XW), jnp.float32),
        pltpu.SemaphoreType.DMA,
        pltpu.SemaphoreType.DMA,
    ],
    compiler_params=_SC_PARAMS,
)
def _sc_scatter(ee, tt, edges64, outm, outt,
                idxr, be, bt, zbe, zbt, accm, acct, s0, s1):
    cid = lax.axis_index("c")
    sid = lax.axis_index("s")
    wid = sid * NC + cid
    sems = (s0, s1)

    def zrow(i, _):
        r = i // (H // 16)
        o = (i % (H // 16)) * 16
        zbe[r, pl.ds(o, 16)] = jnp.zeros((16,), jnp.float32)
        return 0

    lax.fori_loop(0, ZROWS * (H // 16), zrow, 0)

    def zrowt(i, _):
        zbt[i, pl.ds(0, 16)] = jnp.zeros((16,), jnp.float32)
        return 0

    lax.fori_loop(0, ZROWS, zrowt, 0)

    def zcopy(k, _):
        r0 = sid * NRT + k * ZROWS
        pltpu.sync_copy(zbe, accm.at[pl.ds(r0, ZROWS)])
        pltpu.sync_copy(zbt, acct.at[pl.ds(r0, ZROWS)])
        return 0

    lax.fori_loop(0, NRT // ZROWS, zcopy, 0)

    cb0 = wid * CPS
    base0 = cb0 * CH_S
    pltpu.sync_copy(edges64.at[0, pl.ds(cb0, CPS)], idxr.at[pl.ds(0, CPS)])
    plsc.subcore_barrier()

    def l_descs(u, base):
        return (
            pltpu.make_async_copy(ee.at[pl.ds(base, CH_S)], be.at[u], sems[u]),
            pltpu.make_async_copy(tt.at[pl.ds(base, CH_S)], bt.at[u], sems[u]),
        )

    def s_descs(u, ci):
        return (
            pltpu.make_async_copy(be.at[u], accm.at[idxr.at[ci]], sems[u]),
            pltpu.make_async_copy(bt.at[u], acct.at[idxr.at[ci]], sems[u]),
        )

    def step(u, i):
        @pl.when(jnp.logical_and(i >= 2, i < CPS))
        def _():
            for d in s_descs(u, i - 2):
                d.wait()

        @pl.when(i < CPS)
        def _():
            for d in l_descs(u, base0 + i * CH_S):
                d.start()

        @pl.when(jnp.logical_and(i >= 1, i <= CPS))
        def _():
            for d in l_descs((u - 1) % 2, base0 + (i - 1) * CH_S):
                d.wait()
            for d in s_descs((u - 1) % 2, i - 1):
                d.start(add=True)

    def body(q, _):
        for u in range(2):
            step(u, q * 2 + u)
        return 0

    lax.fori_loop(0, CPS // 2 + 1, body, 0)
    for k in range(2):
        ci = CPS - 2 + k
        for d in s_descs(ci % 2, ci):
            d.wait()

    @pl.when(wid < EXTRA_S)
    def _():
        cbx = CPS * NW + wid
        basex = cbx * CH_S
        pltpu.sync_copy(edges64.at[0, pl.ds(cbx, 1)], idxr.at[pl.ds(CPS, 1)])
        for d in l_descs(0, basex):
            d.start()
        for d in l_descs(0, basex):
            d.wait()
        for d in s_descs(0, CPS):
            d.start(add=True)
        for d in s_descs(0, CPS):
            d.wait()

    plsc.subcore_barrier()

    def drain(k, _):
        r0 = sid * NRT + k * ZROWS
        pltpu.sync_copy(accm.at[pl.ds(r0, ZROWS)],
                        outm.at[cid, pl.ds(r0, ZROWS)])
        pltpu.sync_copy(acct.at[pl.ds(r0, ZROWS)],
                        outt.at[cid, pl.ds(r0, ZROWS)])
        return 0

    lax.fori_loop(0, NRT // ZROWS, drain, 0)


_BE = 3200


def _edge_body(gr, gc, gxr, gxc, ea, w1e, w1r, w2, b2, wc1, bc1, wc2r,
               ee, tt):
    d = gxr[...] - gxc[...]
    rad = jnp.sum(d * d, axis=1, keepdims=True)
    s = gr[...] + gc[...] + rad * w1r[...] + jnp.dot(
        ea[...], w1e[...], preferred_element_type=jnp.float32,
        precision=lax.Precision.DEFAULT)
    e1 = _lrelu(s)
    e2 = _lrelu(jnp.dot(e1, w2[...], preferred_element_type=jnp.float32,
                        precision=lax.Precision.DEFAULT) + b2[...])
    c1 = _lrelu(jnp.dot(e2, wc1[...], preferred_element_type=jnp.float32,
                        precision=lax.Precision.DEFAULT) + bc1[...])
    c = jnp.sum(c1 * wc2r[...], axis=1, keepdims=True)
    t = jnp.clip(d * c, -100.0, 100.0)
    lane = lax.broadcasted_iota(jnp.int32, t.shape, 1)
    t = jnp.where(lane == 3, 1.0, t)
    ee[...] = e2
    tt[...] = t


def _tc_edge(gr, gc, gxr, gxc, ea, w1e, w1r, w2, b2, wc1, bc1, wc2r):
    grid = EH // _BE
    hspec = pl.BlockSpec((_BE, H), lambda i: (i, 0))
    xspec = pl.BlockSpec((_BE, XW), lambda i: (i, 0))
    full = lambda a: pl.BlockSpec(a.shape, lambda i: (0,) * a.ndim)
    return pl.pallas_call(
        _edge_body,
        grid=grid,
        in_specs=[
            hspec, hspec, xspec, xspec,
            pl.BlockSpec((_BE, 16), lambda i: (i, 0)),
            full(w1e), full(w1r), full(w2), full(b2), full(wc1), full(bc1),
            full(wc2r),
        ],
        out_specs=[hspec, xspec],
        out_shape=[
            jax.ShapeDtypeStruct((EH, H), jnp.float32),
            jax.ShapeDtypeStruct((EH, XW), jnp.float32),
        ],
        compiler_params=pltpu.CompilerParams(
            dimension_semantics=("arbitrary",)),
    )(gr, gc, gxr, gxc, ea, w1e, w1r, w2, b2, wc1, bc1, wc2r)


_BN = 2000


def _dot(a, b):
    return jnp.dot(a, b, preferred_element_type=jnp.float32,
                   precision=lax.Precision.DEFAULT)


def _init_body(h, xp, we, be, w1a, w1b, b1, hout, rout, cout, xout):
    h0 = _dot(h[...], we[...]) + be[...]
    hout[...] = h0
    rout[...] = _dot(h0, w1a[...]) + b1[...]
    cout[...] = _dot(h0, w1b[...])
    xout[...] = xp[...]


def _tc_init(h, xp, we, be, w1a, w1b, b1):
    grid = N // _BN
    full = lambda a: pl.BlockSpec(a.shape, lambda i: (0,) * a.ndim)
    hspec = pl.BlockSpec((_BN, H), lambda i: (i, 0))
    xspec = pl.BlockSpec((_BN, XW), lambda i: (i, 0))
    return pl.pallas_call(
        _init_body,
        grid=grid,
        in_specs=[hspec, xspec, full(we), full(be), full(w1a), full(w1b),
                  full(b1)],
        out_specs=[hspec, hspec, hspec, xspec],
        out_shape=[
            jax.ShapeDtypeStruct((N, H), jnp.float32),
            jax.ShapeDtypeStruct((N, H), jnp.float32),
            jax.ShapeDtypeStruct((N, H), jnp.float32),
            jax.ShapeDtypeStruct((N, XW), jnp.float32),
        ],
        compiler_params=pltpu.CompilerParams(
            dimension_semantics=("arbitrary",)),
    )(h, xp, we, be, w1a, w1b, b1)


def _node_common(h, xold, pma, pta, pmb, ptb):
    m = (pma[0] + pma[1]) + (pmb[0] + pmb[1])
    tacc = (pta[0] + pta[1]) + (ptb[0] + ptb[1])
    cnt = tacc[:, 3:4]
    scale = 1.0 / jnp.maximum(cnt, 1.0)
    lane = lax.broadcasted_iota(jnp.int32, tacc.shape, 1)
    agg = jnp.where(lane < 3, tacc * scale, 0.0)
    return m, xold + agg


def _node_body(h, xold, pma, pta, pmb, ptb, wn1a, wn1b, bn1, wn2, bn2,
               w1a, w1b, b1, hout, rout, cout, xout):
    m, xnew = _node_common(h[...], xold[...], pma[...], pta[...],
                           pmb[...], ptb[...])
    nh1 = _lrelu(_dot(h[...], wn1a[...]) + _dot(m, wn1b[...]) + bn1[...])
    hnew = h[...] + _dot(nh1, wn2[...]) + bn2[...]
    hout[...] = hnew
    rout[...] = _dot(hnew, w1a[...]) + b1[...]
    cout[...] = _dot(hnew, w1b[...])
    xout[...] = xnew


def _tc_node(h, xold, pma, pta, pmb, ptb, wn1a, wn1b, bn1, wn2, bn2,
             w1a, w1b, b1):
    grid = N // _BN
    full = lambda a: pl.BlockSpec(a.shape, lambda i: (0,) * a.ndim)
    hspec = pl.BlockSpec((_BN, H), lambda i: (i, 0))
    xspec = pl.BlockSpec((_BN, XW), lambda i: (i, 0))
    return pl.pallas_call(
        _node_body,
        grid=grid,
        in_specs=[
            hspec, xspec,
            pl.BlockSpec((NC, _BN, H), lambda i: (0, i, 0)),
            pl.BlockSpec((NC, _BN, XW), lambda i: (0, i, 0)),
            pl.BlockSpec((NC, _BN, H), lambda i: (0, i, 0)),
            pl.BlockSpec((NC, _BN, XW), lambda i: (0, i, 0)),
            full(wn1a), full(wn1b), full(bn1), full(wn2), full(bn2),
            full(w1a), full(w1b), full(b1),
        ],
        out_specs=[hspec, hspec, hspec, xspec],
        out_shape=[
            jax.ShapeDtypeStruct((N, H), jnp.float32),
            jax.ShapeDtypeStruct((N, H), jnp.float32),
            jax.ShapeDtypeStruct((N, H), jnp.float32),
            jax.ShapeDtypeStruct((N, XW), jnp.float32),
        ],
        compiler_params=pltpu.CompilerParams(
            dimension_semantics=("arbitrary",)),
    )(h, xold, pma, pta, pmb, ptb, wn1a, wn1b, bn1, wn2, bn2, w1a, w1b, b1)


def _node_last_body(h, xold, pma, pta, pmb, ptb, wn1a, wn1b, bn1, wn2, bn2,
                    wp, bp, hout, xout):
    m, xnew = _node_common(h[...], xold[...], pma[...], pta[...],
                           pmb[...], ptb[...])
    nh1 = _lrelu(_dot(h[...], wn1a[...]) + _dot(m, wn1b[...]) + bn1[...])
    hnew = h[...] + _dot(nh1, wn2[...]) + bn2[...]
    hout[...] = _dot(hnew, wp[...]) + bp[...]
    xout[...] = xnew


def _tc_node_last(h, xold, pma, pta, pmb, ptb, wn1a, wn1b, bn1, wn2, bn2,
                  wp, bp):
    grid = N // _BN
    full = lambda a: pl.BlockSpec(a.shape, lambda i: (0,) * a.ndim)
    hspec = pl.BlockSpec((_BN, H), lambda i: (i, 0))
    xspec = pl.BlockSpec((_BN, XW), lambda i: (i, 0))
    return pl.pallas_call(
        _node_last_body,
        grid=grid,
        in_specs=[
            hspec, xspec,
            pl.BlockSpec((NC, _BN, H), lambda i: (0, i, 0)),
            pl.BlockSpec((NC, _BN, XW), lambda i: (0, i, 0)),
            pl.BlockSpec((NC, _BN, H), lambda i: (0, i, 0)),
            pl.BlockSpec((NC, _BN, XW), lambda i: (0, i, 0)),
            full(wn1a), full(wn1b), full(bn1), full(wn2), full(bn2),
            full(wp), full(bp),
        ],
        out_specs=[
            pl.BlockSpec((_BN, PROJ), lambda i: (i, 0)),
            xspec,
        ],
        out_shape=[
            jax.ShapeDtypeStruct((N, PROJ), jnp.float32),
            jax.ShapeDtypeStruct((N, XW), jnp.float32),
        ],
        compiler_params=pltpu.CompilerParams(
            dimension_semantics=("arbitrary",)),
    )(h, xold, pma, pta, pmb, ptb, wn1a, wn1b, bn1, wn2, bn2, wp, bp)


def kernel(h, x, edges, vel, edge_attr, params):
    edges = edges.astype(jnp.int32)
    e3 = edges.reshape(2, E // CHUNK, CHUNK)
    e64 = edges.reshape(2, E // CH_S, CH_S)
    e3h = (e3[:, :EH // CHUNK], e3[:, EH // CHUNK:])
    e64h = (e64[:, :EH // CH_S], e64[:, EH // CH_S:])
    eah = (edge_attr[:EH], edge_attr[EH:])
    xp = jnp.pad(x.astype(jnp.float32), ((0, 0), (0, XW - 3)))

    def row(v):
        return v.reshape(1, -1)

    lw = []
    for lp in params["layers"]:
        w1 = lp["edge_mlp"][0]["w"]
        lw.append(dict(
            w1a=w1[:H], w1b=w1[H:2 * H], w1r=w1[2 * H:2 * H + 1],
            w1e=w1[2 * H + 1:], b1=row(lp["edge_mlp"][0]["b"]),
            w2=lp["edge_mlp"][1]["w"], b2=row(lp["edge_mlp"][1]["b"]),
            wc1=lp["coord_mlp"][0]["w"], bc1=row(lp["coord_mlp"][0]["b"]),
            wc2r=lp["coord_mlp"][1]["w"].reshape(1, H),
            wn1a=lp["node_mlp"][0]["w"][:H], wn1b=lp["node_mlp"][0]["w"][H:],
            bn1=row(lp["node_mlp"][0]["b"]),
            wn2=lp["node_mlp"][1]["w"], bn2=row(lp["node_mlp"][1]["b"]),
        ))

    hcur, rtab, ctab, xcur = _tc_init(
        h, xp, params["embed"]["w"], row(params["embed"]["b"]),
        lw[0]["w1a"], lw[0]["w1b"], lw[0]["b1"])

    for l in range(len(lw)):
        w = lw[l]
        gA = _sc_gather(rtab, ctab, xcur, e3h[0])
        eeA, ttA = _tc_edge(gA[0], gA[1], gA[2], gA[3], eah[0], w["w1e"],
                            w["w1r"], w["w2"], w["b2"], w["wc1"], w["bc1"],
                            w["wc2r"])
        gB = _sc_gather(rtab, ctab, xcur, e3h[1])
        pmA, ptA = _sc_scatter(eeA, ttA, e64h[0])
        eeB, ttB = _tc_edge(gB[0], gB[1], gB[2], gB[3], eah[1], w["w1e"],
                            w["w1r"], w["w2"], w["b2"], w["wc1"], w["bc1"],
                            w["wc2r"])
        pmB, ptB = _sc_scatter(eeB, ttB, e64h[1])
        if l + 1 < len(lw):
            nw = lw[l + 1]
            hcur, rtab, ctab, xcur = _tc_node(
                hcur, xcur, pmA, ptA, pmB, ptB, w["wn1a"], w["wn1b"],
                w["bn1"], w["wn2"], w["bn2"], nw["w1a"], nw["w1b"], nw["b1"])
        else:
            hp, xout = _tc_node_last(
                hcur, xcur, pmA, ptA, pmB, ptB, w["wn1a"], w["wn1b"],
                w["bn1"], w["wn2"], w["bn2"], params["proj"]["w"],
                row(params["proj"]["b"]))

    return (hp, xout[:, :3], vel)


# speedup vs baseline: 3.9338x; 1.0005x over previous
import functools

import jax
import jax.numpy as jnp
from jax import lax
from jax.experimental import pallas as pl
from jax.experimental.pallas import tpu as pltpu
from jax.experimental.pallas import tpu_sc as plsc

N = 10000
E = 320000
H = 128
XW = 16
PROJ = 64

NC = 2
NS = 16
NW = NC * NS
EH = E // 2
CHUNK = 128
CHUNKS_PER_W = EH // (NW * CHUNK)
BASE_COVER = NW * CHUNKS_PER_W * CHUNK
EXTRA_CHUNKS = (EH - BASE_COVER) // CHUNK

ZROWS = 25
NRT = N // NS
CH_S = 64
CPS = EH // (NW * CH_S)
COVER_S = NW * CPS * CH_S
EXTRA_S = (EH - COVER_S) // CH_S

_MESH = plsc.VectorSubcoreMesh(
    core_axis_name="c", subcore_axis_name="s", num_cores=NC, num_subcores=NS
)
_SC_PARAMS = pltpu.CompilerParams(use_tc_tiling_on_sc=False)


def _lrelu(v):
    return jnp.where(v >= 0.0, v, 0.2 * v)


GS = 6


def _gather_stream(tabh, tabx, idx2d, outh, outx, base0, bh, bx, sems):
    def g_descs(u, ci):
        return (
            pltpu.make_async_copy(tabh.at[idx2d.at[ci]], bh.at[u], sems[u]),
            pltpu.make_async_copy(tabx.at[idx2d.at[ci]], bx.at[u], sems[u]),
        )

    def w_descs(u, base):
        return (
            pltpu.make_async_copy(bh.at[u], outh.at[pl.ds(base, CHUNK)], sems[u]),
            pltpu.make_async_copy(bx.at[u], outx.at[pl.ds(base, CHUNK)], sems[u]),
        )

    def step(u, i):
        @pl.when(jnp.logical_and(i >= GS, i < CHUNKS_PER_W))
        def _():
            for d in w_descs(u, base0 + (i - GS) * CHUNK):
                d.wait()

        @pl.when(i < CHUNKS_PER_W)
        def _():
            for d in g_descs(u, i):
                d.start()

        @pl.when(jnp.logical_and(i >= 1, i <= CHUNKS_PER_W))
        def _():
            for d in g_descs((u - 1) % GS, i - 1):
                d.wait()
            for d in w_descs((u - 1) % GS, base0 + (i - 1) * CHUNK):
                d.start()

    def body(q, _):
        for u in range(GS):
            step(u, q * GS + u)
        return 0

    lax.fori_loop(0, CHUNKS_PER_W // GS + 1, body, 0)
    for k in range(GS):
        ci = CHUNKS_PER_W - GS + k
        for d in w_descs(ci % GS, base0 + ci * CHUNK):
            d.wait()


@functools.partial(
    pl.kernel,
    mesh=_MESH,
    out_type=[
        jax.ShapeDtypeStruct((EH, H), jnp.float32),
        jax.ShapeDtypeStruct((EH, H), jnp.float32),
        jax.ShapeDtypeStruct((EH, XW), jnp.float32),
        jax.ShapeDtypeStruct((EH, XW), jnp.float32),
    ],
    scratch_types=[
        pltpu.VMEM((CHUNKS_PER_W + 1, CHUNK), jnp.int32),
        pltpu.VMEM((CHUNKS_PER_W + 1, CHUNK), jnp.int32),
        pltpu.VMEM((GS, CHUNK, H), jnp.float32),
        pltpu.VMEM((GS, CHUNK, XW), jnp.float32),
        pltpu.SemaphoreType.DMA,
        pltpu.SemaphoreType.DMA,
        pltpu.SemaphoreType.DMA,
        pltpu.SemaphoreType.DMA,
        pltpu.SemaphoreType.DMA,
        pltpu.SemaphoreType.DMA,
    ],
    compiler_params=_SC_PARAMS,
)
def _sc_gather(rtab, ctab, xtab, edges3, gr, gc, gxr, gxc,
               idxr, idxc, bh, bx, s0, s1, s2, s3, s4, s5):
    cid = lax.axis_index("c")
    sid = lax.axis_index("s")
    wid = sid * NC + cid
    sems = (s0, s1, s2, s3, s4, s5)

    cb0 = wid * CHUNKS_PER_W
    base0 = cb0 * CHUNK
    pltpu.sync_copy(edges3.at[0, pl.ds(cb0, CHUNKS_PER_W)],
                    idxr.at[pl.ds(0, CHUNKS_PER_W)])
    pltpu.sync_copy(edges3.at[1, pl.ds(cb0, CHUNKS_PER_W)],
                    idxc.at[pl.ds(0, CHUNKS_PER_W)])

    _gather_stream(rtab, xtab, idxr, gr, gxr, base0, bh, bx, sems)
    _gather_stream(ctab, xtab, idxc, gc, gxc, base0, bh, bx, sems)

    @pl.when(wid < EXTRA_CHUNKS)
    def _():
        cbx = BASE_COVER // CHUNK + wid
        basex = cbx * CHUNK
        pltpu.sync_copy(edges3.at[0, pl.ds(cbx, 1)],
                        idxr.at[pl.ds(CHUNKS_PER_W, 1)])
        pltpu.sync_copy(edges3.at[1, pl.ds(cbx, 1)],
                        idxc.at[pl.ds(CHUNKS_PER_W, 1)])
        for tab, idx, outh, outx in (
            (rtab, idxr, gr, gxr), (ctab, idxc, gc, gxc)):
            g0 = pltpu.make_async_copy(tab.at[idx.at[CHUNKS_PER_W]],
                                       bh.at[0], s0)
            g1 = pltpu.make_async_copy(xtab.at[idx.at[CHUNKS_PER_W]],
                                       bx.at[0], s0)
            g0.start(); g1.start(); g0.wait(); g1.wait()
            pltpu.sync_copy(bh.at[0], outh.at[pl.ds(basex, CHUNK)])
            pltpu.sync_copy(bx.at[0], outx.at[pl.ds(basex, CHUNK)])


@functools.partial(
    pl.kernel,
    mesh=_MESH,
    out_type=[
        jax.ShapeDtypeStruct((NC, N, H), jnp.float32),
        jax.ShapeDtypeStruct((NC, N, XW), jnp.float32),
    ],
    scratch_types=[
        pltpu.VMEM((CPS + 1, CH_S), jnp.int32),
        pltpu.VMEM((2, CH_S, H), jnp.float32),
        pltpu.VMEM((2, CH_S, XW), jnp.float32),
        pltpu.VMEM((ZROWS, H), jnp.float32),
        pltpu.VMEM((ZROWS, XW), jnp.float32),
        pltpu.VMEM_SHARED((N, H), jnp.float32),
        pltpu.VMEM_SHARED((N, XW), jnp.float32),
        pltpu.SemaphoreType.DMA,
        pltpu.SemaphoreType.DMA,
    ],
    compiler_params=_SC_PARAMS,
)
def _sc_scatter(ee, tt, edges64, outm, outt,
                idxr, be, bt, zbe, zbt, accm, acct, s0, s1):
    cid = lax.axis_index("c")
    sid = lax.axis_index("s")
    wid = sid * NC + cid
    sems = (s0, s1)

    def zrow(i, _):
        r = i // (H // 16)
        o = (i % (H // 16)) * 16
        zbe[r, pl.ds(o, 16)] = jnp.zeros((16,), jnp.float32)
        return 0

    lax.fori_loop(0, ZROWS * (H // 16), zrow, 0)

    def zrowt(i, _):
        zbt[i, pl.ds(0, 16)] = jnp.zeros((16,), jnp.float32)
        return 0

    lax.fori_loop(0, ZROWS, zrowt, 0)

    def zcopy(k, _):
        r0 = sid * NRT + k * ZROWS
        pltpu.sync_copy(zbe, accm.at[pl.ds(r0, ZROWS)])
        pltpu.sync_copy(zbt, acct.at[pl.ds(r0, ZROWS)])
        return 0

    lax.fori_loop(0, NRT // ZROWS, zcopy, 0)

    cb0 = wid * CPS
    base0 = cb0 * CH_S
    pltpu.sync_copy(edges64.at[0, pl.ds(cb0, CPS)], idxr.at[pl.ds(0, CPS)])
    plsc.subcore_barrier()

    def l_descs(u, base):
        return (
            pltpu.make_async_copy(ee.at[pl.ds(base, CH_S)], be.at[u], sems[u]),
            pltpu.make_async_copy(tt.at[pl.ds(base, CH_S)], bt.at[u], sems[u]),
        )

    def s_descs(u, ci):
        return (
            pltpu.make_async_copy(be.at[u], accm.at[idxr.at[ci]], sems[u]),
            pltpu.make_async_copy(bt.at[u], acct.at[idxr.at[ci]], sems[u]),
        )

    def step(u, i):
        @pl.when(jnp.logical_and(i >= 2, i < CPS))
        def _():
            for d in s_descs(u, i - 2):
                d.wait()

        @pl.when(i < CPS)
        def _():
            for d in l_descs(u, base0 + i * CH_S):
                d.start()

        @pl.when(jnp.logical_and(i >= 1, i <= CPS))
        def _():
            for d in l_descs((u - 1) % 2, base0 + (i - 1) * CH_S):
                d.wait()
            for d in s_descs((u - 1) % 2, i - 1):
                d.start(add=True)

    def body(q, _):
        for u in range(2):
            step(u, q * 2 + u)
        return 0

    lax.fori_loop(0, CPS // 2 + 1, body, 0)
    for k in range(2):
        ci = CPS - 2 + k
        for d in s_descs(ci % 2, ci):
            d.wait()

    @pl.when(wid < EXTRA_S)
    def _():
        cbx = CPS * NW + wid
        basex = cbx * CH_S
        pltpu.sync_copy(edges64.at[0, pl.ds(cbx, 1)], idxr.at[pl.ds(CPS, 1)])
        for d in l_descs(0, basex):
            d.start()
        for d in l_descs(0, basex):
            d.wait()
        for d in s_descs(0, CPS):
            d.start(add=True)
        for d in s_descs(0, CPS):
            d.wait()

    plsc.subcore_barrier()

    def drain(k, _):
        r0 = sid * NRT + k * ZROWS
        pltpu.sync_copy(accm.at[pl.ds(r0, ZROWS)],
                        outm.at[cid, pl.ds(r0, ZROWS)])
        pltpu.sync_copy(acct.at[pl.ds(r0, ZROWS)],
                        outt.at[cid, pl.ds(r0, ZROWS)])
        return 0

    lax.fori_loop(0, NRT // ZROWS, drain, 0)


_BE = 3200


def _edge_body(gr, gc, gxr, gxc, ea, w1e, w1r, w2, b2, wc1, bc1, wc2r,
               ee, tt):
    d = gxr[...] - gxc[...]
    rad = jnp.sum(d * d, axis=1, keepdims=True)
    s = gr[...] + gc[...] + rad * w1r[...] + jnp.dot(
        ea[...], w1e[...], preferred_element_type=jnp.float32,
        precision=lax.Precision.DEFAULT)
    e1 = _lrelu(s)
    e2 = _lrelu(jnp.dot(e1, w2[...], preferred_element_type=jnp.float32,
                        precision=lax.Precision.DEFAULT) + b2[...])
    c1 = _lrelu(jnp.dot(e2, wc1[...], preferred_element_type=jnp.float32,
                        precision=lax.Precision.DEFAULT) + bc1[...])
    c = jnp.sum(c1 * wc2r[...], axis=1, keepdims=True)
    t = jnp.clip(d * c, -100.0, 100.0)
    lane = lax.broadcasted_iota(jnp.int32, t.shape, 1)
    t = jnp.where(lane == 3, 1.0, t)
    ee[...] = e2
    tt[...] = t


def _tc_edge(gr, gc, gxr, gxc, ea, w1e, w1r, w2, b2, wc1, bc1, wc2r):
    grid = EH // _BE
    hspec = pl.BlockSpec((_BE, H), lambda i: (i, 0))
    xspec = pl.BlockSpec((_BE, XW), lambda i: (i, 0))
    full = lambda a: pl.BlockSpec(a.shape, lambda i: (0,) * a.ndim)
    return pl.pallas_call(
        _edge_body,
        grid=grid,
        in_specs=[
            hspec, hspec, xspec, xspec,
            pl.BlockSpec((_BE, 16), lambda i: (i, 0)),
            full(w1e), full(w1r), full(w2), full(b2), full(wc1), full(bc1),
            full(wc2r),
        ],
        out_specs=[hspec, xspec],
        out_shape=[
            jax.ShapeDtypeStruct((EH, H), jnp.float32),
            jax.ShapeDtypeStruct((EH, XW), jnp.float32),
        ],
        compiler_params=pltpu.CompilerParams(
            dimension_semantics=("arbitrary",)),
    )(gr, gc, gxr, gxc, ea, w1e, w1r, w2, b2, wc1, bc1, wc2r)


_BN = 2000


def _dot(a, b):
    return jnp.dot(a, b, preferred_element_type=jnp.float32,
                   precision=lax.Precision.DEFAULT)


def _init_body(h, xp, we, be, w1a, w1b, b1, hout, rout, cout, xout):
    h0 = _dot(h[...], we[...]) + be[...]
    hout[...] = h0
    rout[...] = _dot(h0, w1a[...]) + b1[...]
    cout[...] = _dot(h0, w1b[...])
    xout[...] = xp[...]


def _tc_init(h, xp, we, be, w1a, w1b, b1):
    grid = N // _BN
    full = lambda a: pl.BlockSpec(a.shape, lambda i: (0,) * a.ndim)
    hspec = pl.BlockSpec((_BN, H), lambda i: (i, 0))
    xspec = pl.BlockSpec((_BN, XW), lambda i: (i, 0))
    return pl.pallas_call(
        _init_body,
        grid=grid,
        in_specs=[hspec, xspec, full(we), full(be), full(w1a), full(w1b),
                  full(b1)],
        out_specs=[hspec, hspec, hspec, xspec],
        out_shape=[
            jax.ShapeDtypeStruct((N, H), jnp.float32),
            jax.ShapeDtypeStruct((N, H), jnp.float32),
            jax.ShapeDtypeStruct((N, H), jnp.float32),
            jax.ShapeDtypeStruct((N, XW), jnp.float32),
        ],
        compiler_params=pltpu.CompilerParams(
            dimension_semantics=("arbitrary",)),
    )(h, xp, we, be, w1a, w1b, b1)


def _node_common(h, xold, pma, pta, pmb, ptb):
    m = (pma[0] + pma[1]) + (pmb[0] + pmb[1])
    tacc = (pta[0] + pta[1]) + (ptb[0] + ptb[1])
    cnt = tacc[:, 3:4]
    scale = 1.0 / jnp.maximum(cnt, 1.0)
    lane = lax.broadcasted_iota(jnp.int32, tacc.shape, 1)
    agg = jnp.where(lane < 3, tacc * scale, 0.0)
    return m, xold + agg


def _node_body(h, xold, pma, pta, pmb, ptb, wn1a, wn1b, bn1, wn2, bn2,
               w1a, w1b, b1, hout, rout, cout, xout):
    m, xnew = _node_common(h[...], xold[...], pma[...], pta[...],
                           pmb[...], ptb[...])
    nh1 = _lrelu(_dot(h[...], wn1a[...]) + _dot(m, wn1b[...]) + bn1[...])
    hnew = h[...] + _dot(nh1, wn2[...]) + bn2[...]
    hout[...] = hnew
    rout[...] = _dot(hnew, w1a[...]) + b1[...]
    cout[...] = _dot(hnew, w1b[...])
    xout[...] = xnew


def _tc_node(h, xold, pma, pta, pmb, ptb, wn1a, wn1b, bn1, wn2, bn2,
             w1a, w1b, b1):
    grid = N // _BN
    full = lambda a: pl.BlockSpec(a.shape, lambda i: (0,) * a.ndim)
    hspec = pl.BlockSpec((_BN, H), lambda i: (i, 0))
    xspec = pl.BlockSpec((_BN, XW), lambda i: (i, 0))
    return pl.pallas_call(
        _node_body,
        grid=grid,
        in_specs=[
            hspec, xspec,
            pl.BlockSpec((NC, _BN, H), lambda i: (0, i, 0)),
            pl.BlockSpec((NC, _BN, XW), lambda i: (0, i, 0)),
            pl.BlockSpec((NC, _BN, H), lambda i: (0, i, 0)),
            pl.BlockSpec((NC, _BN, XW), lambda i: (0, i, 0)),
            full(wn1a), full(wn1b), full(bn1), full(wn2), full(bn2),
            full(w1a), full(w1b), full(b1),
        ],
        out_specs=[hspec, hspec, hspec, xspec],
        out_shape=[
            jax.ShapeDtypeStruct((N, H), jnp.float32),
            jax.ShapeDtypeStruct((N, H), jnp.float32),
            jax.ShapeDtypeStruct((N, H), jnp.float32),
            jax.ShapeDtypeStruct((N, XW), jnp.float32),
        ],
        compiler_params=pltpu.CompilerParams(
            dimension_semantics=("arbitrary",)),
    )(h, xold, pma, pta, pmb, ptb, wn1a, wn1b, bn1, wn2, bn2, w1a, w1b, b1)


def _node_last_body(h, xold, pma, pta, pmb, ptb, wn1a, wn1b, bn1, wn2, bn2,
                    wp, bp, hout, xout):
    m, xnew = _node_common(h[...], xold[...], pma[...], pta[...],
                           pmb[...], ptb[...])
    nh1 = _lrelu(_dot(h[...], wn1a[...]) + _dot(m, wn1b[...]) + bn1[...])
    hnew = h[...] + _dot(nh1, wn2[...]) + bn2[...]
    hout[...] = _dot(hnew, wp[...]) + bp[...]
    xout[...] = xnew


def _tc_node_last(h, xold, pma, pta, pmb, ptb, wn1a, wn1b, bn1, wn2, bn2,
                  wp, bp):
    grid = N // _BN
    full = lambda a: pl.BlockSpec(a.shape, lambda i: (0,) * a.ndim)
    hspec = pl.BlockSpec((_BN, H), lambda i: (i, 0))
    xspec = pl.BlockSpec((_BN, XW), lambda i: (i, 0))
    return pl.pallas_call(
        _node_last_body,
        grid=grid,
        in_specs=[
            hspec, xspec,
            pl.BlockSpec((NC, _BN, H), lambda i: (0, i, 0)),
            pl.BlockSpec((NC, _BN, XW), lambda i: (0, i, 0)),
            pl.BlockSpec((NC, _BN, H), lambda i: (0, i, 0)),
            pl.BlockSpec((NC, _BN, XW), lambda i: (0, i, 0)),
            full(wn1a), full(wn1b), full(bn1), full(wn2), full(bn2),
            full(wp), full(bp),
        ],
        out_specs=[
            pl.BlockSpec((_BN, PROJ), lambda i: (i, 0)),
            xspec,
        ],
        out_shape=[
            jax.ShapeDtypeStruct((N, PROJ), jnp.float32),
            jax.ShapeDtypeStruct((N, XW), jnp.float32),
        ],
        compiler_params=pltpu.CompilerParams(
            dimension_semantics=("arbitrary",)),
    )(h, xold, pma, pta, pmb, ptb, wn1a, wn1b, bn1, wn2, bn2, wp, bp)


def kernel(h, x, edges, vel, edge_attr, params):
    edges = edges.astype(jnp.int32)
    e3 = edges.reshape(2, E // CHUNK, CHUNK)
    e64 = edges.reshape(2, E // CH_S, CH_S)
    e3h = (e3[:, :EH // CHUNK], e3[:, EH // CHUNK:])
    e64h = (e64[:, :EH // CH_S], e64[:, EH // CH_S:])
    eah = (edge_attr[:EH], edge_attr[EH:])
    xp = jnp.pad(x.astype(jnp.float32), ((0, 0), (0, XW - 3)))

    def row(v):
        return v.reshape(1, -1)

    lw = []
    for lp in params["layers"]:
        w1 = lp["edge_mlp"][0]["w"]
        lw.append(dict(
            w1a=w1[:H], w1b=w1[H:2 * H], w1r=w1[2 * H:2 * H + 1],
            w1e=w1[2 * H + 1:], b1=row(lp["edge_mlp"][0]["b"]),
            w2=lp["edge_mlp"][1]["w"], b2=row(lp["edge_mlp"][1]["b"]),
            wc1=lp["coord_mlp"][0]["w"], bc1=row(lp["coord_mlp"][0]["b"]),
            wc2r=lp["coord_mlp"][1]["w"].reshape(1, H),
            wn1a=lp["node_mlp"][0]["w"][:H], wn1b=lp["node_mlp"][0]["w"][H:],
            bn1=row(lp["node_mlp"][0]["b"]),
            wn2=lp["node_mlp"][1]["w"], bn2=row(lp["node_mlp"][1]["b"]),
        ))

    hcur, rtab, ctab, xcur = _tc_init(
        h, xp, params["embed"]["w"], row(params["embed"]["b"]),
        lw[0]["w1a"], lw[0]["w1b"], lw[0]["b1"])

    for l in range(len(lw)):
        w = lw[l]
        gA = _sc_gather(rtab, ctab, xcur, e3h[0])
        eeA, ttA = _tc_edge(gA[0], gA[1], gA[2], gA[3], eah[0], w["w1e"],
                            w["w1r"], w["w2"], w["b2"], w["wc1"], w["bc1"],
                            w["wc2r"])
        gB = _sc_gather(rtab, ctab, xcur, e3h[1])
        pmA, ptA = _sc_scatter(eeA, ttA, e64h[0])
        eeB, ttB = _tc_edge(gB[0], gB[1], gB[2], gB[3], eah[1], w["w1e"],
                            w["w1r"], w["w2"], w["b2"], w["wc1"], w["bc1"],
                            w["wc2r"])
        pmB, ptB = _sc_scatter(eeB, ttB, e64h[1])
        if l + 1 < len(lw):
            nw = lw[l + 1]
            hcur, rtab, ctab, xcur = _tc_node(
                hcur, xcur, pmA, ptA, pmB, ptB, w["wn1a"], w["wn1b"],
                w["bn1"], w["wn2"], w["bn2"], nw["w1a"], nw["w1b"], nw["b1"])
        else:
            hp, xout = _tc_node_last(
                hcur, xcur, pmA, ptA, pmB, ptB, w["wn1a"], w["wn1b"],
                w["bn1"], w["wn2"], w["bn2"], params["proj"]["w"],
                row(params["proj"]["b"]))

    return (hp, xout[:, :3], vel)
